# Initial kernel scaffold; baseline (speedup 1.0000x reference)
import functools

import jax
import jax.numpy as jnp
from jax import lax
from jax.experimental import pallas as pl
from jax.experimental.pallas import tpu as pltpu
from jax.experimental.pallas import tpu_sc as plsc

_i32 = jnp.int32
_f32 = jnp.float32

_D = 128
_G = 100
_BS = 10
_GP = 128
_N_PAD = 10240
_NW = 32
_CK = 128
_NPT = _N_PAD // 16


def _it16():
    return lax.iota(_i32, 16)


def _rsqrt16(d):
    i = plsc.bitcast(d, _i32)
    y = plsc.bitcast(jnp.full((16,), 0x5F3759DF, _i32) -
                     lax.shift_right_logical(i, 1), _f32)
    for _ in range(4):
        y = y * (1.5 - 0.5 * d * y * y)
    return jnp.where(d > 0.5, y, 0.0)


def _phase_p_body(ep, row_hbm, col_hbm, ew_hbm, batch_hbm, z16_hbm,
                  norm_hbm, cinv_hbm,
                  deg_sh, cnt_sh, dinv_sh,
                  ridx, cidx, val, sc16, ones16,
                  dtile, dchunk, dinv_full, nbuf, ctile, cinvb, cbc):
    cid = lax.axis_index("c")
    sid = lax.axis_index("s")
    it = _it16()
    z16i = jnp.zeros((16,), _i32)

    for c in range(_NPT // 128):
        pltpu.sync_copy(z16_hbm, deg_sh.at[pl.ds(sid * _NPT + c * 128, 128)])
    pltpu.sync_copy(z16_hbm, sc16)
    pltpu.sync_copy(z16_hbm, ones16)
    for j in range(8):
        plsc.store_scatter(ones16, [it + j * 16, z16i], jnp.ones((16,), _f32))

    @pl.when(cid == 0)
    def _():
        pltpu.sync_copy(z16_hbm.at[pl.ds(0, 8)], cnt_sh.at[pl.ds(sid * 8, 8)])

    plsc.subcore_barrier()

    ept = ep // 16

    def deg_chunk(c, _):
        base = pl.multiple_of(sid * ept + c * _CK, _CK)
        pltpu.sync_copy(col_hbm.at[pl.ds(base, _CK)], cidx)
        pltpu.sync_copy(ew_hbm.at[pl.ds(base, _CK)], val)
        for j in range(8):
            plsc.store_scatter(sc16, [it + j * 16, z16i], val[pl.ds(j * 16, 16)])
        pltpu.sync_copy(sc16, deg_sh.at[cidx], add=True)
        return 0

    lax.fori_loop(0, ept // _CK, deg_chunk, 0)

    @pl.when(cid == 0)
    def _():
        def cnt_chunk(c, _):
            base = pl.multiple_of(sid * _NPT + c * _CK, _CK)
            pltpu.sync_copy(batch_hbm.at[pl.ds(base, _CK)], cidx)
            pltpu.sync_copy(ones16, cnt_sh.at[cidx], add=True)
            return 0
        lax.fori_loop(0, _NPT // _CK, cnt_chunk, 0)

    plsc.subcore_barrier()

    def dinv_chunk(c, _):
        base = pl.multiple_of(sid * _NPT + c * 64, 64)
        pltpu.sync_copy(deg_sh.at[pl.ds(base, 64)], dtile)
        for j in range(4):
            d16 = plsc.load_gather(dtile, [it + j * 16, z16i])
            dchunk[pl.ds(j * 16, 16)] = _rsqrt16(d16)
        pltpu.sync_copy(dchunk, dinv_sh.at[pl.ds(base, 64)])
        return 0

    lax.fori_loop(0, _NPT // 64, dinv_chunk, 0)
    plsc.subcore_barrier()

    pltpu.sync_copy(dinv_sh, dinv_full)
    wid = sid * 2 + cid
    epw = ep // _NW

    def norm_chunk(c, _):
        base = pl.multiple_of(wid * epw + c * _CK, _CK)
        pltpu.sync_copy(row_hbm.at[pl.ds(base, _CK)], ridx)
        pltpu.sync_copy(col_hbm.at[pl.ds(base, _CK)], cidx)
        pltpu.sync_copy(ew_hbm.at[pl.ds(base, _CK)], val)
        for j in range(8):
            r16 = ridx[pl.ds(j * 16, 16)]
            c16 = cidx[pl.ds(j * 16, 16)]
            w16 = val[pl.ds(j * 16, 16)]
            dr = plsc.load_gather(dinv_full, [r16])
            dc = plsc.load_gather(dinv_full, [c16])
            nbuf[pl.ds(j * 16, 16)] = dr * w16 * dc
        pltpu.sync_copy(nbuf, norm_hbm.at[pl.ds(base, _CK)])
        return 0

    lax.fori_loop(0, epw // _CK, norm_chunk, 0)

    @pl.when(jnp.logical_and(cid == 0, sid == 0))
    def _():
        pltpu.sync_copy(cnt_sh, ctile)
        for j in range(8):
            c16 = plsc.load_gather(ctile, [it + j * 16, z16i])
            cinvb[pl.ds(j * 16, 16)] = 1.0 / jnp.maximum(c16, 1.0)

        def bc_row(r, _):
            r16 = jnp.full((16,), r, _i32)
            s16 = plsc.load_gather(cinvb, [r16])
            for v in range(8):
                plsc.store_scatter(cbc, [r16, it + v * 16], s16)
            return 0

        lax.fori_loop(0, _GP, bc_row, 0)
        pltpu.sync_copy(cbc, cinv_hbm)


def _prop_body(ep, pool, xw_hbm, row_hbm, col_hbm, norm_hbm, z128_hbm,
               batch_hbm, out_hbm, acc_sh, pool_sh,
               ridx, cidx, nrm, msg):
    cid = lax.axis_index("c")
    sid = lax.axis_index("s")
    it = _it16()

    for c in range(_NPT // 128):
        pltpu.sync_copy(z128_hbm, acc_sh.at[pl.ds(sid * _NPT + c * 128, 128)])
    if pool:
        @pl.when(sid == 0)
        def _():
            pltpu.sync_copy(z128_hbm, pool_sh)
    plsc.subcore_barrier()

    wid = sid * 2 + cid
    epw = ep // _NW

    def chunk(c, _):
        base = pl.multiple_of(wid * epw + c * _CK, _CK)
        pltpu.sync_copy(row_hbm.at[pl.ds(base, _CK)], ridx)
        pltpu.sync_copy(col_hbm.at[pl.ds(base, _CK)], cidx)
        pltpu.sync_copy(norm_hbm.at[pl.ds(base, _CK)], nrm)
        pltpu.sync_copy(xw_hbm.at[ridx], msg)

        def scale(e, _):
            e16 = jnp.full((16,), e, _i32)
            s16 = plsc.load_gather(nrm, [e16])
            for v in range(8):
                cv = it + v * 16
                m = plsc.load_gather(msg, [e16, cv])
                plsc.store_scatter(msg, [e16, cv], m * s16)
            return 0

        lax.fori_loop(0, _CK, scale, 0)
        pltpu.sync_copy(msg, acc_sh.at[cidx], add=True)
        return 0

    lax.fori_loop(0, epw // _CK, chunk, 0)
    plsc.subcore_barrier()

    if not pool:
        for c in range(_NPT // 128):
            base = sid * _NPT + c * 128
            pltpu.sync_copy(acc_sh.at[pl.ds(base, 128)], msg)
            pltpu.sync_copy(msg, out_hbm.at[cid, pl.ds(base, 128)])
    else:
        for c in range(_NPT // 128):
            base = sid * _NPT + c * 128
            pltpu.sync_copy(acc_sh.at[pl.ds(base, 128)], msg)
            pltpu.sync_copy(batch_hbm.at[pl.ds(base, _CK)], cidx)
            pltpu.sync_copy(msg, pool_sh.at[cidx], add=True)
        plsc.subcore_barrier()

        @pl.when(sid == 0)
        def _():
            pltpu.sync_copy(pool_sh, msg)
            pltpu.sync_copy(msg, out_hbm.at[cid])


def _mm_first_body(x_ref, w_ref, o_ref):
    o_ref[...] = jnp.dot(x_ref[...], w_ref[...], preferred_element_type=_f32)


def _mm_mid_body(p_ref, b_ref, w_ref, o_ref):
    h = jnp.maximum(p_ref[0] + p_ref[1] + b_ref[...], 0.0)
    o_ref[...] = jnp.dot(h, w_ref[...], preferred_element_type=_f32)


def _combine_body(p_ref, cinv_ref, b_ref, o_ref):
    o_ref[...] = (p_ref[0] + p_ref[1]) * cinv_ref[...] + b_ref[...]


@functools.lru_cache(maxsize=None)
def _build(n, e, d):
    ep = -((e + n) // -(_NW * _CK)) * (_NW * _CK)
    mesh = plsc.VectorSubcoreMesh(core_axis_name="c", subcore_axis_name="s")

    phase_p = pl.kernel(
        functools.partial(_phase_p_body, ep),
        out_type=(jax.ShapeDtypeStruct((ep,), _f32),
                  jax.ShapeDtypeStruct((_GP, _D), _f32)),
        mesh=mesh,
        scratch_types=[
            pltpu.VMEM_SHARED((_N_PAD, 16), _f32),
            pltpu.VMEM_SHARED((_GP, 16), _f32),
            pltpu.VMEM_SHARED((_N_PAD,), _f32),
            pltpu.VMEM((_CK,), _i32),
            pltpu.VMEM((_CK,), _i32),
            pltpu.VMEM((_CK,), _f32),
            pltpu.VMEM((_CK, 16), _f32),
            pltpu.VMEM((_CK, 16), _f32),
            pltpu.VMEM((64, 16), _f32),
            pltpu.VMEM((64,), _f32),
            pltpu.VMEM((_N_PAD,), _f32),
            pltpu.VMEM((_CK,), _f32),
            pltpu.VMEM((_GP, 16), _f32),
            pltpu.VMEM((_GP,), _f32),
            pltpu.VMEM((_GP, _D), _f32),
        ],
        name="gcn_phase_p",
    )

    def make_prop(pool):
        return pl.kernel(
            functools.partial(_prop_body, ep, pool),
            out_type=jax.ShapeDtypeStruct(
                (2, _GP, _D) if pool else (2, _N_PAD, _D), _f32),
            mesh=mesh,
            scratch_types=[
                pltpu.VMEM_SHARED((_N_PAD, _D), _f32),
                pltpu.VMEM_SHARED((_GP, _D), _f32),
                pltpu.VMEM((_CK,), _i32),
                pltpu.VMEM((_CK,), _i32),
                pltpu.VMEM((_CK,), _f32),
                pltpu.VMEM((_CK, _D), _f32),
            ],
            name="gcn_prop_pool" if pool else "gcn_prop",
        )

    prop = make_prop(False)
    prop_pool = make_prop(True)

    nb = _N_PAD // 2048
    mm_first = pl.pallas_call(
        _mm_first_body,
        grid=(nb,),
        in_specs=[pl.BlockSpec((2048, d), lambda i: (i, 0)),
                  pl.BlockSpec((d, d), lambda i: (0, 0))],
        out_specs=pl.BlockSpec((2048, d), lambda i: (i, 0)),
        out_shape=jax.ShapeDtypeStruct((_N_PAD, d), _f32),
    )
    mm_mid = pl.pallas_call(
        _mm_mid_body,
        grid=(nb,),
        in_specs=[pl.BlockSpec((2, 2048, d), lambda i: (0, i, 0)),
                  pl.BlockSpec((1, d), lambda i: (0, 0)),
                  pl.BlockSpec((d, d), lambda i: (0, 0))],
        out_specs=pl.BlockSpec((2048, d), lambda i: (i, 0)),
        out_shape=jax.ShapeDtypeStruct((_N_PAD, d), _f32),
    )
    combine = pl.pallas_call(
        _combine_body,
        out_shape=jax.ShapeDtypeStruct((_GP, _D), _f32),
    )
    return ep, phase_p, prop, prop_pool, mm_first, mm_mid, combine


def kernel(x, edge_index, edge_weight, batch, batch_size,
           W_in, b_in, W_h0, b_h0, W_h1, b_h1, W_out, b_out):
    n, d = x.shape
    e = edge_index.shape[1]
    ep, phase_p, prop, prop_pool, mm_first, mm_mid, combine = _build(n, e, d)

    ar = jnp.arange(n, dtype=_i32)
    pad_e = ep - e - n
    row_all = jnp.concatenate([edge_index[0].astype(_i32), ar,
                               jnp.zeros((pad_e,), _i32)])
    col_all = jnp.concatenate([edge_index[1].astype(_i32), ar,
                               jnp.zeros((pad_e,), _i32)])
    ew_all = jnp.concatenate([edge_weight, jnp.ones((n,), _f32),
                              jnp.zeros((pad_e,), _f32)])
    batch_p = jnp.concatenate([batch.astype(_i32),
                               jnp.full((_N_PAD - n,), _GP - 1, _i32)])
    x_p = jnp.zeros((_N_PAD, d), _f32).at[:n].set(x)
    z16 = jnp.zeros((128, 16), _f32)
    z128 = jnp.zeros((128, _D), _f32)

    norm_all, cinv = phase_p(row_all, col_all, ew_all, batch_p, z16)

    xw = mm_first(x_p, W_in)
    p = prop(xw, row_all, col_all, norm_all, z128, batch_p)
    xw = mm_mid(p, b_in.reshape(1, d), W_h0)
    p = prop(xw, row_all, col_all, norm_all, z128, batch_p)
    xw = mm_mid(p, b_h0.reshape(1, d), W_h1)
    p = prop(xw, row_all, col_all, norm_all, z128, batch_p)
    xw = mm_mid(p, b_h1.reshape(1, d), W_out)
    pool = prop_pool(xw, row_all, col_all, norm_all, z128, batch_p)

    out = combine(pool, cinv, b_out.reshape(1, d))
    emb = out[:_G].reshape(_BS, _G // _BS, d)
    return emb + (jnp.asarray(batch_size) * 0).astype(emb.dtype)


# speedup vs baseline: 4.1457x; 4.1457x over previous
import functools

import jax
import jax.numpy as jnp
from jax import lax
from jax.experimental import pallas as pl
from jax.experimental.pallas import tpu as pltpu
from jax.experimental.pallas import tpu_sc as plsc

_i32 = jnp.int32
_f32 = jnp.float32

_D = 128
_G = 100
_BS = 10
_GP = 128
_N_PAD = 10240
_NW = 32
_CK = 128
_NPT = _N_PAD // 16


def _it16():
    return lax.iota(_i32, 16)


def _rsqrt16(d):
    i = plsc.bitcast(d, _i32)
    y = plsc.bitcast(jnp.full((16,), 0x5F3759DF, _i32) -
                     lax.shift_right_logical(i, 1), _f32)
    for _ in range(4):
        y = y * (1.5 - 0.5 * d * y * y)
    return jnp.where(d > 0.5, y, 0.0)


def _phase_p_body(ep, row_hbm, col_hbm, ew_hbm, batch_hbm, z1_hbm,
                  norm_hbm, cinv_hbm,
                  deg_sh, cnt_sh, dinv_sh,
                  ridx, cidx, val, ones,
                  dtile, dchunk, dinv_full, nbuf, ctile, cinvb, cbc):
    cid = lax.axis_index("c")
    sid = lax.axis_index("s")

    pltpu.sync_copy(z1_hbm, val)
    for c in range(_NPT // 128):
        pltpu.sync_copy(val, deg_sh.at[pl.ds(sid * _NPT + c * 128, 128)])
    for j in range(8):
        ones[pl.ds(j * 16, 16)] = jnp.ones((16,), _f32)

    @pl.when(jnp.logical_and(cid == 0, sid < 8))
    def _():
        pltpu.sync_copy(val.at[pl.ds(0, 16)], cnt_sh.at[pl.ds(sid * 16, 16)])

    plsc.subcore_barrier()

    ept = ep // 16

    def deg_chunk(c, _):
        base = pl.multiple_of(sid * ept + c * _CK, _CK)
        pltpu.sync_copy(col_hbm.at[pl.ds(base, _CK)], cidx)
        pltpu.sync_copy(ew_hbm.at[pl.ds(base, _CK)], val)
        pltpu.sync_copy(val, deg_sh.at[cidx], add=True)
        return 0

    lax.fori_loop(0, ept // _CK, deg_chunk, 0)

    @pl.when(cid == 0)
    def _():
        def cnt_chunk(c, _):
            base = pl.multiple_of(sid * _NPT + c * _CK, _CK)
            pltpu.sync_copy(batch_hbm.at[pl.ds(base, _CK)], cidx)
            pltpu.sync_copy(ones, cnt_sh.at[cidx], add=True)
            return 0
        lax.fori_loop(0, _NPT // _CK, cnt_chunk, 0)

    plsc.subcore_barrier()

    def dinv_chunk(c, _):
        base = pl.multiple_of(sid * _NPT + c * 64, 64)
        pltpu.sync_copy(deg_sh.at[pl.ds(base, 64)], dtile)
        for j in range(4):
            d16 = dtile[pl.ds(j * 16, 16)]
            dchunk[pl.ds(j * 16, 16)] = _rsqrt16(d16)
        pltpu.sync_copy(dchunk, dinv_sh.at[pl.ds(base, 64)])
        return 0

    lax.fori_loop(0, _NPT // 64, dinv_chunk, 0)
    plsc.subcore_barrier()

    pltpu.sync_copy(dinv_sh, dinv_full)
    wid = sid * 2 + cid
    epw = ep // _NW

    def norm_chunk(c, _):
        base = pl.multiple_of(wid * epw + c * _CK, _CK)
        pltpu.sync_copy(row_hbm.at[pl.ds(base, _CK)], ridx)
        pltpu.sync_copy(col_hbm.at[pl.ds(base, _CK)], cidx)
        pltpu.sync_copy(ew_hbm.at[pl.ds(base, _CK)], val)
        for j in range(8):
            r16 = ridx[pl.ds(j * 16, 16)]
            c16 = cidx[pl.ds(j * 16, 16)]
            w16 = val[pl.ds(j * 16, 16)]
            dr = plsc.load_gather(dinv_full, [r16])
            dc = plsc.load_gather(dinv_full, [c16])
            nbuf[pl.ds(j * 16, 16)] = dr * w16 * dc
        pltpu.sync_copy(nbuf, norm_hbm.at[pl.ds(base, _CK)])
        return 0

    lax.fori_loop(0, epw // _CK, norm_chunk, 0)

    @pl.when(jnp.logical_and(cid == 0, sid == 0))
    def _():
        pltpu.sync_copy(cnt_sh, ctile)
        for j in range(8):
            c16 = ctile[pl.ds(j * 16, 16)]
            cinvb[pl.ds(j * 16, 16)] = 1.0 / jnp.maximum(c16, 1.0)

        def bc_row(r, _):
            s16 = plsc.load_gather(cinvb, [jnp.full((16,), r, _i32)])
            for v in range(8):
                cbc[pl.ds(r * _D + v * 16, 16)] = s16
            return 0

        lax.fori_loop(0, _GP, bc_row, 0)
        pltpu.sync_copy(cbc, cinv_hbm)


def _prop_body(ep, pool, xw_hbm, row_hbm, col_hbm, norm_hbm, z128_hbm,
               batch_hbm, out_hbm, acc_sh, pool_sh,
               ridx, cidx, nrm, msg):
    cid = lax.axis_index("c")
    sid = lax.axis_index("s")
    it = _it16()

    pltpu.sync_copy(z128_hbm, msg)
    for c in range(_NPT // 128):
        pltpu.sync_copy(msg, acc_sh.at[pl.ds(sid * _NPT + c * 128, 128)])
    if pool:
        @pl.when(sid == 0)
        def _():
            pltpu.sync_copy(msg, pool_sh)
    plsc.subcore_barrier()

    wid = sid * 2 + cid
    epw = ep // _NW

    def chunk(c, _):
        base = pl.multiple_of(wid * epw + c * _CK, _CK)
        pltpu.sync_copy(row_hbm.at[pl.ds(base, _CK)], ridx)
        pltpu.sync_copy(col_hbm.at[pl.ds(base, _CK)], cidx)
        pltpu.sync_copy(norm_hbm.at[pl.ds(base, _CK)], nrm)
        pltpu.sync_copy(xw_hbm.at[ridx], msg)

        def scale(e, _):
            e16 = jnp.full((16,), e, _i32)
            s16 = plsc.load_gather(nrm, [e16])
            for v in range(8):
                cv = it + v * 16
                m = plsc.load_gather(msg, [e16, cv])
                plsc.store_scatter(msg, [e16, cv], m * s16)
            return 0

        lax.fori_loop(0, _CK, scale, 0)
        pltpu.sync_copy(msg, acc_sh.at[cidx], add=True)
        return 0

    lax.fori_loop(0, epw // _CK, chunk, 0)
    plsc.subcore_barrier()

    if not pool:
        for c in range(_NPT // 128):
            base = sid * _NPT + c * 128
            pltpu.sync_copy(acc_sh.at[pl.ds(base, 128)], msg)
            pltpu.sync_copy(msg, out_hbm.at[cid, pl.ds(base, 128)])
    else:
        for c in range(_NPT // 128):
            base = sid * _NPT + c * 128
            pltpu.sync_copy(acc_sh.at[pl.ds(base, 128)], msg)
            pltpu.sync_copy(batch_hbm.at[pl.ds(base, _CK)], cidx)
            pltpu.sync_copy(msg, pool_sh.at[cidx], add=True)
        plsc.subcore_barrier()

        @pl.when(sid == 0)
        def _():
            pltpu.sync_copy(pool_sh, msg)
            pltpu.sync_copy(msg, out_hbm.at[cid])


def _mm_first_body(x_ref, w_ref, o_ref):
    o_ref[...] = jnp.dot(x_ref[...], w_ref[...], preferred_element_type=_f32)


def _mm_mid_body(p_ref, b_ref, w_ref, o_ref):
    h = jnp.maximum(p_ref[0] + p_ref[1] + b_ref[...], 0.0)
    o_ref[...] = jnp.dot(h, w_ref[...], preferred_element_type=_f32)


def _combine_body(p_ref, cinv_ref, b_ref, o_ref):
    o_ref[...] = (p_ref[0] + p_ref[1]) * cinv_ref[...] + b_ref[...]


@functools.lru_cache(maxsize=None)
def _build(n, e, d):
    ep = -((e + n) // -(_NW * _CK)) * (_NW * _CK)
    mesh = plsc.VectorSubcoreMesh(core_axis_name="c", subcore_axis_name="s")
    sc_params = pltpu.CompilerParams(needs_layout_passes=False)

    phase_p = pl.kernel(
        functools.partial(_phase_p_body, ep),
        out_type=(jax.ShapeDtypeStruct((ep,), _f32),
                  jax.ShapeDtypeStruct((_GP * _D,), _f32)),
        mesh=mesh,
        scratch_types=[
            pltpu.VMEM_SHARED((_N_PAD,), _f32),
            pltpu.VMEM_SHARED((_GP,), _f32),
            pltpu.VMEM_SHARED((_N_PAD,), _f32),
            pltpu.VMEM((_CK,), _i32),
            pltpu.VMEM((_CK,), _i32),
            pltpu.VMEM((_CK,), _f32),
            pltpu.VMEM((_CK,), _f32),
            pltpu.VMEM((64,), _f32),
            pltpu.VMEM((64,), _f32),
            pltpu.VMEM((_N_PAD,), _f32),
            pltpu.VMEM((_CK,), _f32),
            pltpu.VMEM((_GP,), _f32),
            pltpu.VMEM((_GP,), _f32),
            pltpu.VMEM((_GP * _D,), _f32),
        ],
        name="gcn_phase_p",
        compiler_params=sc_params,
    )

    def make_prop(pool):
        return pl.kernel(
            functools.partial(_prop_body, ep, pool),
            out_type=jax.ShapeDtypeStruct(
                (2, _GP, _D) if pool else (2, _N_PAD, _D), _f32),
            mesh=mesh,
            scratch_types=[
                pltpu.VMEM_SHARED((_N_PAD, _D), _f32),
                pltpu.VMEM_SHARED((_GP, _D), _f32),
                pltpu.VMEM((_CK,), _i32),
                pltpu.VMEM((_CK,), _i32),
                pltpu.VMEM((_CK,), _f32),
                pltpu.VMEM((_CK, _D), _f32),
            ],
            name="gcn_prop_pool" if pool else "gcn_prop",
            compiler_params=sc_params,
        )

    prop = make_prop(False)
    prop_pool = make_prop(True)

    nb = _N_PAD // 2048
    mm_first = pl.pallas_call(
        _mm_first_body,
        grid=(nb,),
        in_specs=[pl.BlockSpec((2048, d), lambda i: (i, 0)),
                  pl.BlockSpec((d, d), lambda i: (0, 0))],
        out_specs=pl.BlockSpec((2048, d), lambda i: (i, 0)),
        out_shape=jax.ShapeDtypeStruct((_N_PAD, d), _f32),
    )
    mm_mid = pl.pallas_call(
        _mm_mid_body,
        grid=(nb,),
        in_specs=[pl.BlockSpec((2, 2048, d), lambda i: (0, i, 0)),
                  pl.BlockSpec((1, d), lambda i: (0, 0)),
                  pl.BlockSpec((d, d), lambda i: (0, 0))],
        out_specs=pl.BlockSpec((2048, d), lambda i: (i, 0)),
        out_shape=jax.ShapeDtypeStruct((_N_PAD, d), _f32),
    )
    combine = pl.pallas_call(
        _combine_body,
        out_shape=jax.ShapeDtypeStruct((_GP, _D), _f32),
    )
    return ep, phase_p, prop, prop_pool, mm_first, mm_mid, combine


def kernel(x, edge_index, edge_weight, batch, batch_size,
           W_in, b_in, W_h0, b_h0, W_h1, b_h1, W_out, b_out):
    n, d = x.shape
    e = edge_index.shape[1]
    ep, phase_p, prop, prop_pool, mm_first, mm_mid, combine = _build(n, e, d)

    ar = jnp.arange(n, dtype=_i32)
    pad_e = ep - e - n
    row_all = jnp.concatenate([edge_index[0].astype(_i32), ar,
                               jnp.zeros((pad_e,), _i32)])
    col_all = jnp.concatenate([edge_index[1].astype(_i32), ar,
                               jnp.zeros((pad_e,), _i32)])
    ew_all = jnp.concatenate([edge_weight, jnp.ones((n,), _f32),
                              jnp.zeros((pad_e,), _f32)])
    batch_p = jnp.concatenate([batch.astype(_i32),
                               jnp.full((_N_PAD - n,), _GP - 1, _i32)])
    x_p = jnp.zeros((_N_PAD, d), _f32).at[:n].set(x)
    z1 = jnp.zeros((128,), _f32)
    z128 = jnp.zeros((128, _D), _f32)

    norm_all, cinv = phase_p(row_all, col_all, ew_all, batch_p, z1)

    xw = mm_first(x_p, W_in)
    p = prop(xw, row_all, col_all, norm_all, z128, batch_p)
    xw = mm_mid(p, b_in.reshape(1, d), W_h0)
    p = prop(xw, row_all, col_all, norm_all, z128, batch_p)
    xw = mm_mid(p, b_h0.reshape(1, d), W_h1)
    p = prop(xw, row_all, col_all, norm_all, z128, batch_p)
    xw = mm_mid(p, b_h1.reshape(1, d), W_out)
    pool = prop_pool(xw, row_all, col_all, norm_all, z128, batch_p)

    out = combine(pool, cinv.reshape(_GP, _D), b_out.reshape(1, d))
    emb = out[:_G].reshape(_BS, _G // _BS, d)
    return emb + (jnp.asarray(batch_size) * 0).astype(emb.dtype)


# speedup vs baseline: 9.9569x; 2.4017x over previous
import functools

import jax
import jax.numpy as jnp
from jax import lax
from jax.experimental import pallas as pl
from jax.experimental.pallas import tpu as pltpu
from jax.experimental.pallas import tpu_sc as plsc

_i32 = jnp.int32
_f32 = jnp.float32

_D = 128
_G = 100
_BS = 10
_GP = 128
_N_PAD = 10240
_NW = 32
_CK = 112
_NPT = _N_PAD // 16
_BCK = 64


def _it16():
    return lax.iota(_i32, 16)


def _rsqrt16(d):
    i = plsc.bitcast(d, _i32)
    y = plsc.bitcast(jnp.full((16,), 0x5F3759DF, _i32) -
                     lax.shift_right_logical(i, 1), _f32)
    for _ in range(4):
        y = y * (1.5 - 0.5 * d * y * y)
    return jnp.where(d > 0.5, y, 0.0)


def _zero_chunks():
    off, out = 0, []
    while off < _NPT:
        sz = min(_CK, _NPT - off)
        out.append((off, sz))
        off += sz
    return out


def _phase_p_body(ep, row_hbm, col_hbm, ew_hbm, batch_hbm, z1_hbm,
                  norm_hbm, cinv_hbm,
                  deg_sh, cnt_sh, dinv_sh,
                  ewb_p, colb_p, rowb_n, colb_n, ewb_n,
                  ones, pidx, dtile, dchunk, dinv_full, nbuf0, nbuf1,
                  ctile, cinvb, cbc, dsem, ns0, ns1):
    cid = lax.axis_index("c")
    sid = lax.axis_index("s")

    pltpu.sync_copy(z1_hbm, nbuf0)
    for off, sz in _zero_chunks():
        pltpu.sync_copy(nbuf0.at[pl.ds(0, sz)],
                        deg_sh.at[pl.ds(sid * _NPT + off, sz)])
    for j in range(_CK // 16):
        ones[pl.ds(j * 16, 16)] = jnp.ones((16,), _f32)

    @pl.when(jnp.logical_and(cid == 0, sid < 8))
    def _():
        pltpu.sync_copy(nbuf0.at[pl.ds(0, 16)], cnt_sh.at[pl.ds(sid * 16, 16)])

    plsc.subcore_barrier()

    npc = ep // 16 // _CK
    pltpu.sync_copy(ew_hbm.at[pl.ds(sid * npc, npc)], ewb_p)
    pltpu.sync_copy(col_hbm.at[pl.ds(sid * npc, npc)], colb_p)

    def _dstart(c):
        pltpu.async_copy(ewb_p.at[c], deg_sh.at[colb_p.at[c]], dsem, add=True)

    def _dwait():
        pltpu.make_async_copy(ewb_p.at[0], deg_sh.at[colb_p.at[0]],
                              dsem).wait()

    _dstart(0)
    _dstart(1)

    def deg_chunk(c, _):
        _dwait()
        _dstart(c + 2)
        return 0

    lax.fori_loop(0, npc - 2, deg_chunk, 0)
    _dwait()
    _dwait()

    @pl.when(cid == 0)
    def _():
        for k in range(_NPT // _BCK):
            pltpu.sync_copy(batch_hbm.at[sid * (_NPT // _BCK) + k], pidx)
            pltpu.sync_copy(ones.at[pl.ds(0, _BCK)], cnt_sh.at[pidx],
                            add=True)

    plsc.subcore_barrier()

    def dinv_chunk(c, _):
        base = pl.multiple_of(sid * _NPT + c * 64, 64)
        pltpu.sync_copy(deg_sh.at[pl.ds(base, 64)], dtile)
        for j in range(4):
            d16 = dtile[pl.ds(j * 16, 16)]
            dchunk[pl.ds(j * 16, 16)] = _rsqrt16(d16)
        pltpu.sync_copy(dchunk, dinv_sh.at[pl.ds(base, 64)])
        return 0

    lax.fori_loop(0, _NPT // 64, dinv_chunk, 0)
    plsc.subcore_barrier()

    pltpu.sync_copy(dinv_sh, dinv_full)
    wid = sid * 2 + cid
    nc = ep // _NW // _CK
    pltpu.sync_copy(row_hbm.at[pl.ds(wid * nc, nc)], rowb_n)
    pltpu.sync_copy(col_hbm.at[pl.ds(wid * nc, nc)], colb_n)
    pltpu.sync_copy(ew_hbm.at[pl.ds(wid * nc, nc)], ewb_n)

    def _ncompute(c, buf):
        for j in range(_CK // 16):
            r16 = rowb_n[c, pl.ds(j * 16, 16)]
            c16 = colb_n[c, pl.ds(j * 16, 16)]
            w16 = ewb_n[c, pl.ds(j * 16, 16)]
            dr = plsc.load_gather(dinv_full, [r16])
            dc = plsc.load_gather(dinv_full, [c16])
            buf[pl.ds(j * 16, 16)] = dr * w16 * dc

    def _nstart(c, buf, sem):
        pltpu.async_copy(buf, norm_hbm.at[wid * nc + c], sem)

    def _nwait(buf, sem):
        pltpu.make_async_copy(buf, norm_hbm.at[0], sem).wait()

    def npair(cc, _):
        for par in range(2):
            c = 2 * cc + par
            buf, sem = (nbuf0, ns0) if par == 0 else (nbuf1, ns1)

            @pl.when(cc >= 1)
            def _(buf=buf, sem=sem):
                _nwait(buf, sem)

            _ncompute(c, buf)
            _nstart(c, buf, sem)
        return 0

    lax.fori_loop(0, (nc - 1) // 2, npair, 0)
    _nwait(nbuf0, ns0)
    _ncompute(nc - 1, nbuf0)
    _nstart(nc - 1, nbuf0, ns0)
    _nwait(nbuf0, ns0)
    _nwait(nbuf1, ns1)

    @pl.when(jnp.logical_and(cid == 0, sid == 0))
    def _():
        pltpu.sync_copy(cnt_sh, ctile)
        for j in range(8):
            c16 = ctile[pl.ds(j * 16, 16)]
            cinvb[pl.ds(j * 16, 16)] = 1.0 / jnp.maximum(c16, 1.0)

        def bc_row(r, _):
            s16 = plsc.load_gather(cinvb, [jnp.full((16,), r, _i32)])
            for v in range(8):
                cbc[pl.ds(r * _D + v * 16, 16)] = s16
            return 0

        lax.fori_loop(0, _GP, bc_row, 0)
        pltpu.sync_copy(cbc, cinv_hbm)


def _prop_body(ep, pool, xw_hbm, row_hbm, col_hbm, norm_hbm, z_hbm,
               batch_hbm, out_hbm, acc_sh,
               rb0, rb1, rb2, cb0, cb1, cb2, nb0, nb1, nb2,
               m0, m1, m2, pidx,
               is0, is1, is2, gs0, gs1, gs2, ss0, ss1, ss2, *maybe_pool):
    pool_sh = maybe_pool[0] if pool else None
    cid = lax.axis_index("c")
    sid = lax.axis_index("s")

    pltpu.sync_copy(z_hbm, m0)
    for off, sz in _zero_chunks():
        pltpu.sync_copy(m0.at[pl.ds(0, sz)],
                        acc_sh.at[pl.ds(sid * _NPT + off, sz)])
    if pool:
        @pl.when(sid == 0)
        def _():
            pltpu.sync_copy(m0, pool_sh.at[pl.ds(0, _CK)])
            pltpu.sync_copy(m0.at[pl.ds(0, _GP - _CK)],
                            pool_sh.at[pl.ds(_CK, _GP - _CK)])

    wid = sid * 2 + cid
    nc = ep // _NW // _CK
    cbase = wid * nc
    slots = ((rb0, cb0, nb0, m0, is0, gs0, ss0),
             (rb1, cb1, nb1, m1, is1, gs1, ss1),
             (rb2, cb2, nb2, m2, is2, gs2, ss2))

    def _istart(c, s):
        rb, cb, nb = s[0], s[1], s[2]
        pltpu.async_copy(row_hbm.at[cbase + c], rb, s[4])
        pltpu.async_copy(col_hbm.at[cbase + c], cb, s[4])
        pltpu.async_copy(norm_hbm.at[cbase + c], nb, s[4])

    def _iwait(s):
        for _ in range(3):
            pltpu.make_async_copy(row_hbm.at[0], s[0], s[4]).wait()

    def _gstart(c, s):
        pltpu.async_copy(xw_hbm.at[s[0]], s[3], s[5])

    def _gwait(s):
        pltpu.make_async_copy(xw_hbm.at[s[0]], s[3], s[5]).wait()

    def _sstart(c, s):
        pltpu.async_copy(s[3], acc_sh.at[s[1]], s[6], add=True)

    def _swait(s):
        pltpu.make_async_copy(s[3], acc_sh.at[s[1]], s[6]).wait()

    def _scale(s):
        nb, m = s[2], s[3]

        def sc(e, _):
            s16 = plsc.load_gather(nb, [jnp.full((16,), e, _i32)])
            for v in range(8):
                x = m[e, pl.ds(v * 16, 16)]
                m[e, pl.ds(v * 16, 16)] = x * s16
            return 0

        lax.fori_loop(0, _CK, sc, 0)

    plsc.subcore_barrier()

    _istart(0, slots[0])
    _istart(1, slots[1])
    _istart(2, slots[2])
    _iwait(slots[0])
    _gstart(0, slots[0])

    def triple(cc, _):
        for par in range(3):
            c = cc * 3 + par
            s = slots[par]
            sg = slots[(par + 1) % 3]
            sf = slots[(par + 2) % 3]
            _gwait(s)

            @pl.when(c <= nc - 2)
            def _(c=c, sg=sg):
                _iwait(sg)
                _gstart(c + 1, sg)

            _scale(s)

            @pl.when(jnp.logical_and(c >= 1, c <= nc - 3))
            def _(c=c, sf=sf):
                _swait(sf)
                _istart(c + 2, sf)

            _sstart(c, s)
        return 0

    lax.fori_loop(0, nc // 3, triple, 0)
    for s in slots:
        _swait(s)
    plsc.subcore_barrier()

    if not pool:
        for off, sz in _zero_chunks():
            base = sid * _NPT + off
            pltpu.sync_copy(acc_sh.at[pl.ds(base, sz)], m0.at[pl.ds(0, sz)])
            pltpu.sync_copy(m0.at[pl.ds(0, sz)],
                            out_hbm.at[cid, pl.ds(base, sz)])
    else:
        for k in range(_NPT // _BCK):
            base = sid * _NPT + k * _BCK
            pltpu.sync_copy(acc_sh.at[pl.ds(base, _BCK)],
                            m0.at[pl.ds(0, _BCK)])
            pltpu.sync_copy(batch_hbm.at[sid * (_NPT // _BCK) + k], pidx)
            pltpu.sync_copy(m0.at[pl.ds(0, _BCK)], pool_sh.at[pidx],
                            add=True)
        plsc.subcore_barrier()

        @pl.when(sid == 0)
        def _():
            pltpu.sync_copy(pool_sh.at[pl.ds(0, _CK)], m0)
            pltpu.sync_copy(m0, out_hbm.at[cid, pl.ds(0, _CK)])
            pltpu.sync_copy(pool_sh.at[pl.ds(_CK, _GP - _CK)],
                            m0.at[pl.ds(0, _GP - _CK)])
            pltpu.sync_copy(m0.at[pl.ds(0, _GP - _CK)],
                            out_hbm.at[cid, pl.ds(_CK, _GP - _CK)])


def _mm_first_body(x_ref, w_ref, o_ref):
    o_ref[...] = jnp.dot(x_ref[...], w_ref[...], preferred_element_type=_f32)


def _mm_mid_body(p_ref, b_ref, w_ref, o_ref):
    h = jnp.maximum(p_ref[0] + p_ref[1] + b_ref[...], 0.0)
    o_ref[...] = jnp.dot(h, w_ref[...], preferred_element_type=_f32)


def _combine_body(p_ref, cinv_ref, b_ref, o_ref):
    o_ref[...] = (p_ref[0] + p_ref[1]) * cinv_ref[...] + b_ref[...]


@functools.lru_cache(maxsize=None)
def _build(n, e, d):
    ep = -((e + n) // -(_NW * _CK)) * (_NW * _CK)
    nc = ep // _NW // _CK
    assert nc % 3 == 0 and nc % 2 == 1, nc
    mesh = plsc.VectorSubcoreMesh(core_axis_name="c", subcore_axis_name="s")
    sc_params = pltpu.CompilerParams(needs_layout_passes=False,
                                     use_tc_tiling_on_sc=False)

    phase_p = pl.kernel(
        functools.partial(_phase_p_body, ep),
        out_type=(jax.ShapeDtypeStruct((ep // _CK, _CK), _f32),
                  jax.ShapeDtypeStruct((_GP * _D,), _f32)),
        mesh=mesh,
        scratch_types=[
            pltpu.VMEM_SHARED((_N_PAD,), _f32),
            pltpu.VMEM_SHARED((_GP,), _f32),
            pltpu.VMEM_SHARED((_N_PAD,), _f32),
            pltpu.VMEM((ep // 16 // _CK, _CK), _f32),
            pltpu.VMEM((ep // 16 // _CK, _CK), _i32),
            pltpu.VMEM((ep // _NW // _CK, _CK), _i32),
            pltpu.VMEM((ep // _NW // _CK, _CK), _i32),
            pltpu.VMEM((ep // _NW // _CK, _CK), _f32),
            pltpu.VMEM((_CK,), _f32),
            pltpu.VMEM((_BCK,), _i32),
            pltpu.VMEM((64,), _f32),
            pltpu.VMEM((64,), _f32),
            pltpu.VMEM((_N_PAD,), _f32),
            pltpu.VMEM((_CK,), _f32),
            pltpu.VMEM((_CK,), _f32),
            pltpu.VMEM((_GP,), _f32),
            pltpu.VMEM((_GP,), _f32),
            pltpu.VMEM((_GP * _D,), _f32),
            pltpu.SemaphoreType.DMA,
            pltpu.SemaphoreType.DMA,
            pltpu.SemaphoreType.DMA,
        ],
        name="gcn_phase_p",
        compiler_params=sc_params,
    )

    def make_prop(pool):
        return pl.kernel(
            functools.partial(_prop_body, ep, pool),
            out_type=jax.ShapeDtypeStruct(
                (2, _GP, _D) if pool else (2, _N_PAD, _D), _f32),
            mesh=mesh,
            scratch_types=[
                pltpu.VMEM_SHARED((_N_PAD, _D), _f32),
                pltpu.VMEM((_CK,), _i32),
                pltpu.VMEM((_CK,), _i32),
                pltpu.VMEM((_CK,), _i32),
                pltpu.VMEM((_CK,), _i32),
                pltpu.VMEM((_CK,), _i32),
                pltpu.VMEM((_CK,), _i32),
                pltpu.VMEM((_CK,), _f32),
                pltpu.VMEM((_CK,), _f32),
                pltpu.VMEM((_CK,), _f32),
                pltpu.VMEM((_CK, _D), _f32),
                pltpu.VMEM((_CK, _D), _f32),
                pltpu.VMEM((_CK, _D), _f32),
                pltpu.VMEM((_BCK,), _i32),
                pltpu.SemaphoreType.DMA,
                pltpu.SemaphoreType.DMA,
                pltpu.SemaphoreType.DMA,
                pltpu.SemaphoreType.DMA,
                pltpu.SemaphoreType.DMA,
                pltpu.SemaphoreType.DMA,
                pltpu.SemaphoreType.DMA,
                pltpu.SemaphoreType.DMA,
                pltpu.SemaphoreType.DMA,
            ] + ([pltpu.VMEM_SHARED((_GP, _D), _f32)] if pool else []),
            name="gcn_prop_pool" if pool else "gcn_prop",
            compiler_params=sc_params,
        )

    prop = make_prop(False)
    prop_pool = make_prop(True)

    nb = _N_PAD // 2048
    mm_first = pl.pallas_call(
        _mm_first_body,
        grid=(nb,),
        in_specs=[pl.BlockSpec((2048, d), lambda i: (i, 0)),
                  pl.BlockSpec((d, d), lambda i: (0, 0))],
        out_specs=pl.BlockSpec((2048, d), lambda i: (i, 0)),
        out_shape=jax.ShapeDtypeStruct((_N_PAD, d), _f32),
    )
    mm_mid = pl.pallas_call(
        _mm_mid_body,
        grid=(nb,),
        in_specs=[pl.BlockSpec((2, 2048, d), lambda i: (0, i, 0)),
                  pl.BlockSpec((1, d), lambda i: (0, 0)),
                  pl.BlockSpec((d, d), lambda i: (0, 0))],
        out_specs=pl.BlockSpec((2048, d), lambda i: (i, 0)),
        out_shape=jax.ShapeDtypeStruct((_N_PAD, d), _f32),
    )
    combine = pl.pallas_call(
        _combine_body,
        out_shape=jax.ShapeDtypeStruct((_GP, _D), _f32),
    )
    return ep, phase_p, prop, prop_pool, mm_first, mm_mid, combine


def kernel(x, edge_index, edge_weight, batch, batch_size,
           W_in, b_in, W_h0, b_h0, W_h1, b_h1, W_out, b_out):
    n, d = x.shape
    e = edge_index.shape[1]
    ep, phase_p, prop, prop_pool, mm_first, mm_mid, combine = _build(n, e, d)

    ar = jnp.arange(n, dtype=_i32)
    pad_e = ep - e - n
    row2d = jnp.concatenate([edge_index[0].astype(_i32), ar,
                             jnp.zeros((pad_e,), _i32)]).reshape(-1, _CK)
    col2d = jnp.concatenate([edge_index[1].astype(_i32), ar,
                             jnp.zeros((pad_e,), _i32)]).reshape(-1, _CK)
    ew2d = jnp.concatenate([edge_weight, jnp.ones((n,), _f32),
                            jnp.zeros((pad_e,), _f32)]).reshape(-1, _CK)
    batch2d = jnp.concatenate([batch.astype(_i32),
                               jnp.full((_N_PAD - n,), _GP - 1, _i32)
                               ]).reshape(-1, _BCK)
    x_p = jnp.zeros((_N_PAD, d), _f32).at[:n].set(x)
    z1 = jnp.zeros((_CK,), _f32)
    z = jnp.zeros((_CK, _D), _f32)

    norm2d, cinv = phase_p(row2d, col2d, ew2d, batch2d, z1)

    xw = mm_first(x_p, W_in)
    p = prop(xw, row2d, col2d, norm2d, z, batch2d)
    xw = mm_mid(p, b_in.reshape(1, d), W_h0)
    p = prop(xw, row2d, col2d, norm2d, z, batch2d)
    xw = mm_mid(p, b_h0.reshape(1, d), W_h1)
    p = prop(xw, row2d, col2d, norm2d, z, batch2d)
    xw = mm_mid(p, b_h1.reshape(1, d), W_out)
    pool = prop_pool(xw, row2d, col2d, norm2d, z, batch2d)

    out = combine(pool, cinv.reshape(_GP, _D), b_out.reshape(1, d))
    emb = out[:_G].reshape(_BS, _G // _BS, d)
    return emb + (jnp.asarray(batch_size) * 0).astype(emb.dtype)


# speedup vs baseline: 10.3922x; 1.0437x over previous
import functools

import jax
import jax.numpy as jnp
from jax import lax
from jax.experimental import pallas as pl
from jax.experimental.pallas import tpu as pltpu
from jax.experimental.pallas import tpu_sc as plsc

_i32 = jnp.int32
_f32 = jnp.float32

_D = 128
_G = 100
_BS = 10
_GP = 128
_N_PAD = 10240
_NW = 32
_CK = 112
_NPT = _N_PAD // 16
_BCK = 64


def _it16():
    return lax.iota(_i32, 16)


def _rsqrt16(d):
    i = plsc.bitcast(d, _i32)
    y = plsc.bitcast(jnp.full((16,), 0x5F3759DF, _i32) -
                     lax.shift_right_logical(i, 1), _f32)
    for _ in range(4):
        y = y * (1.5 - 0.5 * d * y * y)
    return jnp.where(d > 0.5, y, 0.0)


def _zero_chunks():
    off, out = 0, []
    while off < _NPT:
        sz = min(_CK, _NPT - off)
        out.append((off, sz))
        off += sz
    return out


def _phase_p_body(ep, row_hbm, col_hbm, ew_hbm, batch_hbm, z1_hbm,
                  norm_hbm, cinv_hbm,
                  deg_sh, cnt_sh, dinv_sh,
                  ewb_p, colb_p, rowb_n, colb_n, ewb_n,
                  ones, pidx, dtile, dchunk, dinv_full, nbuf0, nbuf1,
                  ctile, cinvb, cbc, dsem, ns0, ns1):
    cid = lax.axis_index("c")
    sid = lax.axis_index("s")

    pltpu.sync_copy(z1_hbm, nbuf0)
    for off, sz in _zero_chunks():
        pltpu.sync_copy(nbuf0.at[pl.ds(0, sz)],
                        deg_sh.at[pl.ds(sid * _NPT + off, sz)])
    for j in range(_CK // 16):
        ones[pl.ds(j * 16, 16)] = jnp.ones((16,), _f32)

    @pl.when(jnp.logical_and(cid == 0, sid < 8))
    def _():
        pltpu.sync_copy(nbuf0.at[pl.ds(0, 16)], cnt_sh.at[pl.ds(sid * 16, 16)])

    plsc.subcore_barrier()

    npc = ep // 16 // _CK
    pltpu.sync_copy(ew_hbm.at[pl.ds(sid * npc, npc)], ewb_p)
    pltpu.sync_copy(col_hbm.at[pl.ds(sid * npc, npc)], colb_p)

    def _dstart(c):
        pltpu.async_copy(ewb_p.at[c], deg_sh.at[colb_p.at[c]], dsem, add=True)

    def _dwait():
        pltpu.make_async_copy(ewb_p.at[0], deg_sh.at[colb_p.at[0]],
                              dsem).wait()

    _dstart(0)
    _dstart(1)

    def deg_chunk(c, _):
        _dwait()
        _dstart(c + 2)
        return 0

    lax.fori_loop(0, npc - 2, deg_chunk, 0)
    _dwait()
    _dwait()

    @pl.when(cid == 0)
    def _():
        for k in range(_NPT // _BCK):
            pltpu.sync_copy(batch_hbm.at[sid * (_NPT // _BCK) + k], pidx)
            pltpu.sync_copy(ones.at[pl.ds(0, _BCK)], cnt_sh.at[pidx],
                            add=True)

    plsc.subcore_barrier()

    def dinv_chunk(c, _):
        base = pl.multiple_of(sid * _NPT + c * 64, 64)
        pltpu.sync_copy(deg_sh.at[pl.ds(base, 64)], dtile)
        for j in range(4):
            d16 = dtile[pl.ds(j * 16, 16)]
            dchunk[pl.ds(j * 16, 16)] = _rsqrt16(d16)
        pltpu.sync_copy(dchunk, dinv_sh.at[pl.ds(base, 64)])
        return 0

    lax.fori_loop(0, _NPT // 64, dinv_chunk, 0)
    plsc.subcore_barrier()

    pltpu.sync_copy(dinv_sh, dinv_full)
    wid = sid * 2 + cid
    nc = ep // _NW // _CK
    pltpu.sync_copy(row_hbm.at[pl.ds(wid * nc, nc)], rowb_n)
    pltpu.sync_copy(col_hbm.at[pl.ds(wid * nc, nc)], colb_n)
    pltpu.sync_copy(ew_hbm.at[pl.ds(wid * nc, nc)], ewb_n)

    def _ncompute(c, buf):
        for j in range(_CK // 16):
            r16 = rowb_n[c, pl.ds(j * 16, 16)]
            c16 = colb_n[c, pl.ds(j * 16, 16)]
            w16 = ewb_n[c, pl.ds(j * 16, 16)]
            dr = plsc.load_gather(dinv_full, [r16])
            dc = plsc.load_gather(dinv_full, [c16])
            buf[pl.ds(j * 16, 16)] = dr * w16 * dc

    def _nstart(c, buf, sem):
        pltpu.async_copy(buf, norm_hbm.at[wid * nc + c], sem)

    def _nwait(buf, sem):
        pltpu.make_async_copy(buf, norm_hbm.at[0], sem).wait()

    def npair(cc, _):
        for par in range(2):
            c = 2 * cc + par
            buf, sem = (nbuf0, ns0) if par == 0 else (nbuf1, ns1)

            @pl.when(cc >= 1)
            def _(buf=buf, sem=sem):
                _nwait(buf, sem)

            _ncompute(c, buf)
            _nstart(c, buf, sem)
        return 0

    lax.fori_loop(0, (nc - 1) // 2, npair, 0)
    _nwait(nbuf0, ns0)
    _ncompute(nc - 1, nbuf0)
    _nstart(nc - 1, nbuf0, ns0)
    _nwait(nbuf0, ns0)
    _nwait(nbuf1, ns1)

    @pl.when(jnp.logical_and(cid == 0, sid == 0))
    def _():
        pltpu.sync_copy(cnt_sh, ctile)
        for j in range(8):
            c16 = ctile[pl.ds(j * 16, 16)]
            cinvb[pl.ds(j * 16, 16)] = 1.0 / jnp.maximum(c16, 1.0)

        def bc_row(r, _):
            s16 = plsc.load_gather(cinvb, [jnp.full((16,), r, _i32)])
            for v in range(8):
                cbc[pl.ds(r * _D + v * 16, 16)] = s16
            return 0

        lax.fori_loop(0, _GP, bc_row, 0)
        pltpu.sync_copy(cbc, cinv_hbm)


def _prop_body(ep, pool, xw_hbm, row_hbm, col_hbm, norm_hbm, z_hbm,
               batch_hbm, out_hbm, acc_sh,
               rb0, rb1, rb2, cb0, cb1, cb2, nb0, nb1, nb2,
               m0, m1, m2, pidx,
               is0, is1, is2, gs0, gs1, gs2, ss0, ss1, ss2, *maybe_pool):
    pool_sh = maybe_pool[0] if pool else None
    cid = lax.axis_index("c")
    sid = lax.axis_index("s")

    pltpu.sync_copy(z_hbm, m0)
    for off, sz in _zero_chunks():
        pltpu.sync_copy(m0.at[pl.ds(0, sz)],
                        acc_sh.at[pl.ds(sid * _NPT + off, sz)])
    if pool:
        @pl.when(sid == 0)
        def _():
            pltpu.sync_copy(m0, pool_sh.at[pl.ds(0, _CK)])
            pltpu.sync_copy(m0.at[pl.ds(0, _GP - _CK)],
                            pool_sh.at[pl.ds(_CK, _GP - _CK)])

    wid = sid * 2 + cid
    nc = ep // _NW // _CK
    cbase = wid * nc
    slots = ((rb0, cb0, nb0, m0, is0, gs0, ss0),
             (rb1, cb1, nb1, m1, is1, gs1, ss1),
             (rb2, cb2, nb2, m2, is2, gs2, ss2))

    def _istart(c, s):
        rb, cb, nb = s[0], s[1], s[2]
        pltpu.async_copy(row_hbm.at[cbase + c], rb, s[4])
        pltpu.async_copy(col_hbm.at[cbase + c], cb, s[4])
        pltpu.async_copy(norm_hbm.at[cbase + c], nb, s[4])

    def _iwait(s):
        for _ in range(3):
            pltpu.make_async_copy(row_hbm.at[0], s[0], s[4]).wait()

    def _gstart(c, s):
        pltpu.async_copy(xw_hbm.at[s[0]], s[3], s[5])

    def _gwait(s):
        pltpu.make_async_copy(xw_hbm.at[s[0]], s[3], s[5]).wait()

    def _sstart(c, s):
        pltpu.async_copy(s[3], acc_sh.at[s[1]], s[6], add=True)

    def _swait(s):
        pltpu.make_async_copy(s[3], acc_sh.at[s[1]], s[6]).wait()

    def _scale(s):
        nb, m = s[2], s[3]

        def sc(i, _):
            e = i * 2
            sa = plsc.load_gather(nb, [jnp.full((16,), e, _i32)])
            sb = plsc.load_gather(nb, [jnp.full((16,), e + 1, _i32)])
            for v in range(8):
                xa = m[e, pl.ds(v * 16, 16)]
                xb = m[e + 1, pl.ds(v * 16, 16)]
                m[e, pl.ds(v * 16, 16)] = xa * sa
                m[e + 1, pl.ds(v * 16, 16)] = xb * sb
            return 0

        lax.fori_loop(0, _CK // 2, sc, 0)

    plsc.subcore_barrier()

    _istart(0, slots[0])
    _istart(1, slots[1])
    _istart(2, slots[2])
    _iwait(slots[0])
    _gstart(0, slots[0])

    def triple(cc, _):
        for par in range(3):
            c = cc * 3 + par
            s = slots[par]
            sg = slots[(par + 1) % 3]
            sf = slots[(par + 2) % 3]
            _gwait(s)

            @pl.when(c <= nc - 2)
            def _(c=c, sg=sg):
                _iwait(sg)
                _gstart(c + 1, sg)

            _scale(s)

            @pl.when(jnp.logical_and(c >= 1, c <= nc - 3))
            def _(c=c, sf=sf):
                _swait(sf)
                _istart(c + 2, sf)

            _sstart(c, s)
        return 0

    lax.fori_loop(0, nc // 3, triple, 0)
    for s in slots:
        _swait(s)
    plsc.subcore_barrier()

    if not pool:
        for off, sz in _zero_chunks():
            base = sid * _NPT + off
            pltpu.sync_copy(acc_sh.at[pl.ds(base, sz)], m0.at[pl.ds(0, sz)])
            pltpu.sync_copy(m0.at[pl.ds(0, sz)],
                            out_hbm.at[cid, pl.ds(base, sz)])
    else:
        for k in range(_NPT // _BCK):
            base = sid * _NPT + k * _BCK
            pltpu.sync_copy(acc_sh.at[pl.ds(base, _BCK)],
                            m0.at[pl.ds(0, _BCK)])
            pltpu.sync_copy(batch_hbm.at[sid * (_NPT // _BCK) + k], pidx)
            pltpu.sync_copy(m0.at[pl.ds(0, _BCK)], pool_sh.at[pidx],
                            add=True)
        plsc.subcore_barrier()

        @pl.when(sid == 0)
        def _():
            pltpu.sync_copy(pool_sh.at[pl.ds(0, _CK)], m0)
            pltpu.sync_copy(m0, out_hbm.at[cid, pl.ds(0, _CK)])
            pltpu.sync_copy(pool_sh.at[pl.ds(_CK, _GP - _CK)],
                            m0.at[pl.ds(0, _GP - _CK)])
            pltpu.sync_copy(m0.at[pl.ds(0, _GP - _CK)],
                            out_hbm.at[cid, pl.ds(_CK, _GP - _CK)])


def _mm_first_body(x_ref, w_ref, o_ref):
    o_ref[...] = jnp.dot(x_ref[...], w_ref[...], preferred_element_type=_f32)


def _mm_mid_body(p_ref, b_ref, w_ref, o_ref):
    h = jnp.maximum(p_ref[0] + p_ref[1] + b_ref[...], 0.0)
    o_ref[...] = jnp.dot(h, w_ref[...], preferred_element_type=_f32)


def _combine_body(p_ref, cinv_ref, b_ref, o_ref):
    o_ref[...] = (p_ref[0] + p_ref[1]) * cinv_ref[...] + b_ref[...]


@functools.lru_cache(maxsize=None)
def _build(n, e, d):
    ep = -((e + n) // -(_NW * _CK)) * (_NW * _CK)
    nc = ep // _NW // _CK
    assert nc % 3 == 0 and nc % 2 == 1, nc
    mesh = plsc.VectorSubcoreMesh(core_axis_name="c", subcore_axis_name="s")
    sc_params = pltpu.CompilerParams(needs_layout_passes=False,
                                     use_tc_tiling_on_sc=False)

    phase_p = pl.kernel(
        functools.partial(_phase_p_body, ep),
        out_type=(jax.ShapeDtypeStruct((ep // _CK, _CK), _f32),
                  jax.ShapeDtypeStruct((_GP * _D,), _f32)),
        mesh=mesh,
        scratch_types=[
            pltpu.VMEM_SHARED((_N_PAD,), _f32),
            pltpu.VMEM_SHARED((_GP,), _f32),
            pltpu.VMEM_SHARED((_N_PAD,), _f32),
            pltpu.VMEM((ep // 16 // _CK, _CK), _f32),
            pltpu.VMEM((ep // 16 // _CK, _CK), _i32),
            pltpu.VMEM((ep // _NW // _CK, _CK), _i32),
            pltpu.VMEM((ep // _NW // _CK, _CK), _i32),
            pltpu.VMEM((ep // _NW // _CK, _CK), _f32),
            pltpu.VMEM((_CK,), _f32),
            pltpu.VMEM((_BCK,), _i32),
            pltpu.VMEM((64,), _f32),
            pltpu.VMEM((64,), _f32),
            pltpu.VMEM((_N_PAD,), _f32),
            pltpu.VMEM((_CK,), _f32),
            pltpu.VMEM((_CK,), _f32),
            pltpu.VMEM((_GP,), _f32),
            pltpu.VMEM((_GP,), _f32),
            pltpu.VMEM((_GP * _D,), _f32),
            pltpu.SemaphoreType.DMA,
            pltpu.SemaphoreType.DMA,
            pltpu.SemaphoreType.DMA,
        ],
        name="gcn_phase_p",
        compiler_params=sc_params,
    )

    def make_prop(pool):
        return pl.kernel(
            functools.partial(_prop_body, ep, pool),
            out_type=jax.ShapeDtypeStruct(
                (2, _GP, _D) if pool else (2, _N_PAD, _D), _f32),
            mesh=mesh,
            scratch_types=[
                pltpu.VMEM_SHARED((_N_PAD, _D), _f32),
                pltpu.VMEM((_CK,), _i32),
                pltpu.VMEM((_CK,), _i32),
                pltpu.VMEM((_CK,), _i32),
                pltpu.VMEM((_CK,), _i32),
                pltpu.VMEM((_CK,), _i32),
                pltpu.VMEM((_CK,), _i32),
                pltpu.VMEM((_CK,), _f32),
                pltpu.VMEM((_CK,), _f32),
                pltpu.VMEM((_CK,), _f32),
                pltpu.VMEM((_CK, _D), _f32),
                pltpu.VMEM((_CK, _D), _f32),
                pltpu.VMEM((_CK, _D), _f32),
                pltpu.VMEM((_BCK,), _i32),
                pltpu.SemaphoreType.DMA,
                pltpu.SemaphoreType.DMA,
                pltpu.SemaphoreType.DMA,
                pltpu.SemaphoreType.DMA,
                pltpu.SemaphoreType.DMA,
                pltpu.SemaphoreType.DMA,
                pltpu.SemaphoreType.DMA,
                pltpu.SemaphoreType.DMA,
                pltpu.SemaphoreType.DMA,
            ] + ([pltpu.VMEM_SHARED((_GP, _D), _f32)] if pool else []),
            name="gcn_prop_pool" if pool else "gcn_prop",
            compiler_params=sc_params,
        )

    prop = make_prop(False)
    prop_pool = make_prop(True)

    nb = _N_PAD // 2048
    mm_first = pl.pallas_call(
        _mm_first_body,
        grid=(nb,),
        in_specs=[pl.BlockSpec((2048, d), lambda i: (i, 0)),
                  pl.BlockSpec((d, d), lambda i: (0, 0))],
        out_specs=pl.BlockSpec((2048, d), lambda i: (i, 0)),
        out_shape=jax.ShapeDtypeStruct((_N_PAD, d), _f32),
    )
    mm_mid = pl.pallas_call(
        _mm_mid_body,
        grid=(nb,),
        in_specs=[pl.BlockSpec((2, 2048, d), lambda i: (0, i, 0)),
                  pl.BlockSpec((1, d), lambda i: (0, 0)),
                  pl.BlockSpec((d, d), lambda i: (0, 0))],
        out_specs=pl.BlockSpec((2048, d), lambda i: (i, 0)),
        out_shape=jax.ShapeDtypeStruct((_N_PAD, d), _f32),
    )
    combine = pl.pallas_call(
        _combine_body,
        out_shape=jax.ShapeDtypeStruct((_GP, _D), _f32),
    )
    return ep, phase_p, prop, prop_pool, mm_first, mm_mid, combine


def kernel(x, edge_index, edge_weight, batch, batch_size,
           W_in, b_in, W_h0, b_h0, W_h1, b_h1, W_out, b_out):
    n, d = x.shape
    e = edge_index.shape[1]
    ep, phase_p, prop, prop_pool, mm_first, mm_mid, combine = _build(n, e, d)

    ar = jnp.arange(n, dtype=_i32)
    pad_e = ep - e - n
    row2d = jnp.concatenate([edge_index[0].astype(_i32), ar,
                             jnp.zeros((pad_e,), _i32)]).reshape(-1, _CK)
    col2d = jnp.concatenate([edge_index[1].astype(_i32), ar,
                             jnp.zeros((pad_e,), _i32)]).reshape(-1, _CK)
    ew2d = jnp.concatenate([edge_weight, jnp.ones((n,), _f32),
                            jnp.zeros((pad_e,), _f32)]).reshape(-1, _CK)
    batch2d = jnp.concatenate([batch.astype(_i32),
                               jnp.full((_N_PAD - n,), _GP - 1, _i32)
                               ]).reshape(-1, _BCK)
    x_p = jnp.zeros((_N_PAD, d), _f32).at[:n].set(x)
    z1 = jnp.zeros((_CK,), _f32)
    z = jnp.zeros((_CK, _D), _f32)

    norm2d, cinv = phase_p(row2d, col2d, ew2d, batch2d, z1)

    xw = mm_first(x_p, W_in)
    p = prop(xw, row2d, col2d, norm2d, z, batch2d)
    xw = mm_mid(p, b_in.reshape(1, d), W_h0)
    p = prop(xw, row2d, col2d, norm2d, z, batch2d)
    xw = mm_mid(p, b_h0.reshape(1, d), W_h1)
    p = prop(xw, row2d, col2d, norm2d, z, batch2d)
    xw = mm_mid(p, b_h1.reshape(1, d), W_out)
    pool = prop_pool(xw, row2d, col2d, norm2d, z, batch2d)

    out = combine(pool, cinv.reshape(_GP, _D), b_out.reshape(1, d))
    emb = out[:_G].reshape(_BS, _G // _BS, d)
    return emb + (jnp.asarray(batch_size) * 0).astype(emb.dtype)


# speedup vs baseline: 11.7214x; 1.1279x over previous
import functools

import jax
import jax.numpy as jnp
from jax import lax
from jax.experimental import pallas as pl
from jax.experimental.pallas import tpu as pltpu
from jax.experimental.pallas import tpu_sc as plsc

_i32 = jnp.int32
_f32 = jnp.float32

_D = 128
_G = 100
_BS = 10
_GP = 128
_N_PAD = 10240
_NW = 32
_CK = 112
_NPT = _N_PAD // 16
_BCK = 64


def _it16():
    return lax.iota(_i32, 16)


def _rsqrt16(d):
    i = plsc.bitcast(d, _i32)
    y = plsc.bitcast(jnp.full((16,), 0x5F3759DF, _i32) -
                     lax.shift_right_logical(i, 1), _f32)
    for _ in range(4):
        y = y * (1.5 - 0.5 * d * y * y)
    return jnp.where(d > 0.5, y, 0.0)


def _zero_chunks():
    off, out = 0, []
    while off < _NPT:
        sz = min(_CK, _NPT - off)
        out.append((off, sz))
        off += sz
    return out


def _phase_p_body(ep, row_hbm, col_hbm, ew_hbm, batch_hbm, z1_hbm,
                  norm_hbm, cinv_hbm,
                  deg_sh, cnt_sh, dinv_sh,
                  ewb_p, colb_p, rowb_n, colb_n, ewb_n,
                  ones, pidx, dtile, dchunk, dinv_full, nbuf0, nbuf1,
                  ctile, cinvb, cbc, dsem, ns0, ns1):
    cid = lax.axis_index("c")
    sid = lax.axis_index("s")

    pltpu.sync_copy(z1_hbm, nbuf0)
    for off, sz in _zero_chunks():
        pltpu.sync_copy(nbuf0.at[pl.ds(0, sz)],
                        deg_sh.at[pl.ds(sid * _NPT + off, sz)])
    for j in range(_CK // 16):
        ones[pl.ds(j * 16, 16)] = jnp.ones((16,), _f32)

    @pl.when(jnp.logical_and(cid == 0, sid < 8))
    def _():
        pltpu.sync_copy(nbuf0.at[pl.ds(0, 16)], cnt_sh.at[pl.ds(sid * 16, 16)])

    plsc.subcore_barrier()

    npc = ep // 16 // _CK
    pltpu.sync_copy(ew_hbm.at[pl.ds(sid * npc, npc)], ewb_p)
    pltpu.sync_copy(col_hbm.at[pl.ds(sid * npc, npc)], colb_p)

    def _dstart(c):
        pltpu.async_copy(ewb_p.at[c], deg_sh.at[colb_p.at[c]], dsem, add=True)

    def _dwait():
        pltpu.make_async_copy(ewb_p.at[0], deg_sh.at[colb_p.at[0]],
                              dsem).wait()

    _dstart(0)
    _dstart(1)

    def deg_chunk(c, _):
        _dwait()
        _dstart(c + 2)
        return 0

    lax.fori_loop(0, npc - 2, deg_chunk, 0)
    _dwait()
    _dwait()

    @pl.when(cid == 0)
    def _():
        for k in range(_NPT // _BCK):
            pltpu.sync_copy(batch_hbm.at[sid * (_NPT // _BCK) + k], pidx)
            pltpu.sync_copy(ones.at[pl.ds(0, _BCK)], cnt_sh.at[pidx],
                            add=True)

    plsc.subcore_barrier()

    def dinv_chunk(c, _):
        base = pl.multiple_of(sid * _NPT + c * 64, 64)
        pltpu.sync_copy(deg_sh.at[pl.ds(base, 64)], dtile)
        for j in range(4):
            d16 = dtile[pl.ds(j * 16, 16)]
            dchunk[pl.ds(j * 16, 16)] = _rsqrt16(d16)
        pltpu.sync_copy(dchunk, dinv_sh.at[pl.ds(base, 64)])
        return 0

    lax.fori_loop(0, _NPT // 64, dinv_chunk, 0)
    plsc.subcore_barrier()

    pltpu.sync_copy(dinv_sh, dinv_full)
    wid = sid * 2 + cid
    nc = ep // _NW // _CK
    pltpu.sync_copy(row_hbm.at[pl.ds(wid * nc, nc)], rowb_n)
    pltpu.sync_copy(col_hbm.at[pl.ds(wid * nc, nc)], colb_n)
    pltpu.sync_copy(ew_hbm.at[pl.ds(wid * nc, nc)], ewb_n)

    def _ncompute(c, buf):
        for j in range(_CK // 16):
            r16 = rowb_n[c, pl.ds(j * 16, 16)]
            c16 = colb_n[c, pl.ds(j * 16, 16)]
            w16 = ewb_n[c, pl.ds(j * 16, 16)]
            dr = plsc.load_gather(dinv_full, [r16])
            dc = plsc.load_gather(dinv_full, [c16])
            buf[pl.ds(j * 16, 16)] = dr * w16 * dc

    def _nstart(c, buf, sem):
        pltpu.async_copy(buf, norm_hbm.at[wid * nc + c], sem)

    def _nwait(buf, sem):
        pltpu.make_async_copy(buf, norm_hbm.at[0], sem).wait()

    def npair(cc, _):
        for par in range(2):
            c = 2 * cc + par
            buf, sem = (nbuf0, ns0) if par == 0 else (nbuf1, ns1)

            @pl.when(cc >= 1)
            def _(buf=buf, sem=sem):
                _nwait(buf, sem)

            _ncompute(c, buf)
            _nstart(c, buf, sem)
        return 0

    lax.fori_loop(0, (nc - 1) // 2, npair, 0)
    _nwait(nbuf0, ns0)
    _ncompute(nc - 1, nbuf0)
    _nstart(nc - 1, nbuf0, ns0)
    _nwait(nbuf0, ns0)
    _nwait(nbuf1, ns1)

    @pl.when(jnp.logical_and(cid == 0, sid == 0))
    def _():
        pltpu.sync_copy(cnt_sh, ctile)
        for j in range(8):
            c16 = ctile[pl.ds(j * 16, 16)]
            cinvb[pl.ds(j * 16, 16)] = 1.0 / jnp.maximum(c16, 1.0)

        def bc_row(r, _):
            s16 = plsc.load_gather(cinvb, [jnp.full((16,), r, _i32)])
            for v in range(8):
                cbc[pl.ds(r * _D + v * 16, 16)] = s16
            return 0

        lax.fori_loop(0, _GP, bc_row, 0)
        pltpu.sync_copy(cbc, cinv_hbm)


def _prop_body(ep, nc0, nc1, pool, xw_hbm, row_hbm, col_hbm, norm_hbm, z_hbm,
               batch_hbm, out_hbm, acc_sh,
               rb0, rb1, rb2, cb0, cb1, cb2, nb0, nb1, nb2,
               m0, m1, m2, pidx,
               is0, is1, is2, gs0, gs1, gs2, ss0, ss1, ss2, *maybe_pool):
    pool_sh = maybe_pool[0] if pool else None
    cid = lax.axis_index("c")
    sid = lax.axis_index("s")

    pltpu.sync_copy(z_hbm, m0)
    for off, sz in _zero_chunks():
        pltpu.sync_copy(m0.at[pl.ds(0, sz)],
                        acc_sh.at[pl.ds(sid * _NPT + off, sz)])
    if pool:
        @pl.when(sid == 0)
        def _():
            pltpu.sync_copy(m0, pool_sh.at[pl.ds(0, _CK)])
            pltpu.sync_copy(m0.at[pl.ds(0, _GP - _CK)],
                            pool_sh.at[pl.ds(_CK, _GP - _CK)])

    nc = jnp.where(cid == 0, nc0, nc1)
    cbase = jnp.where(cid == 0, sid * nc0, 16 * nc0 + sid * nc1)
    slots = ((rb0, cb0, nb0, m0, is0, gs0, ss0),
             (rb1, cb1, nb1, m1, is1, gs1, ss1),
             (rb2, cb2, nb2, m2, is2, gs2, ss2))

    def _istart(c, s):
        rb, cb, nb = s[0], s[1], s[2]
        pltpu.async_copy(row_hbm.at[cbase + c], rb, s[4])
        pltpu.async_copy(col_hbm.at[cbase + c], cb, s[4])
        pltpu.async_copy(norm_hbm.at[cbase + c], nb, s[4])

    def _iwait(s):
        for _ in range(3):
            pltpu.make_async_copy(row_hbm.at[0], s[0], s[4]).wait()

    def _gstart(c, s):
        pltpu.async_copy(xw_hbm.at[s[0]], s[3], s[5])

    def _gwait(s):
        pltpu.make_async_copy(xw_hbm.at[s[0]], s[3], s[5]).wait()

    def _sstart(c, s):
        pltpu.async_copy(s[3], acc_sh.at[s[1]], s[6], add=True)

    def _swait(s):
        pltpu.make_async_copy(s[3], acc_sh.at[s[1]], s[6]).wait()

    def _scale(s):
        nb, m = s[2], s[3]

        def sc(i, _):
            e = i * 2
            sa = plsc.load_gather(nb, [jnp.full((16,), e, _i32)])
            sb = plsc.load_gather(nb, [jnp.full((16,), e + 1, _i32)])
            for v in range(8):
                xa = m[e, pl.ds(v * 16, 16)]
                xb = m[e + 1, pl.ds(v * 16, 16)]
                m[e, pl.ds(v * 16, 16)] = xa * sa
                m[e + 1, pl.ds(v * 16, 16)] = xb * sb
            return 0

        lax.fori_loop(0, _CK // 2, sc, 0)

    plsc.subcore_barrier()

    _istart(0, slots[0])
    _istart(1, slots[1])
    _istart(2, slots[2])
    _iwait(slots[0])
    _gstart(0, slots[0])

    def triple(cc, _):
        for par in range(3):
            c = cc * 3 + par
            s = slots[par]
            sg = slots[(par + 1) % 3]
            sf = slots[(par + 2) % 3]
            _gwait(s)

            @pl.when(c <= nc - 2)
            def _(c=c, sg=sg):
                _iwait(sg)
                _gstart(c + 1, sg)

            _scale(s)

            @pl.when(jnp.logical_and(c >= 1, c <= nc - 3))
            def _(c=c, sf=sf):
                _swait(sf)
                _istart(c + 2, sf)

            _sstart(c, s)
        return 0

    lax.fori_loop(0, nc // 3, triple, 0)
    for s in slots:
        _swait(s)
    plsc.subcore_barrier()

    if not pool:
        for off, sz in _zero_chunks():
            base = sid * _NPT + off
            pltpu.sync_copy(acc_sh.at[pl.ds(base, sz)], m0.at[pl.ds(0, sz)])
            pltpu.sync_copy(m0.at[pl.ds(0, sz)],
                            out_hbm.at[cid, pl.ds(base, sz)])
    else:
        for k in range(_NPT // _BCK):
            base = sid * _NPT + k * _BCK
            pltpu.sync_copy(acc_sh.at[pl.ds(base, _BCK)],
                            m0.at[pl.ds(0, _BCK)])
            pltpu.sync_copy(batch_hbm.at[sid * (_NPT // _BCK) + k], pidx)
            pltpu.sync_copy(m0.at[pl.ds(0, _BCK)], pool_sh.at[pidx],
                            add=True)
        plsc.subcore_barrier()

        @pl.when(sid == 0)
        def _():
            pltpu.sync_copy(pool_sh.at[pl.ds(0, _CK)], m0)
            pltpu.sync_copy(m0, out_hbm.at[cid, pl.ds(0, _CK)])
            pltpu.sync_copy(pool_sh.at[pl.ds(_CK, _GP - _CK)],
                            m0.at[pl.ds(0, _GP - _CK)])
            pltpu.sync_copy(m0.at[pl.ds(0, _GP - _CK)],
                            out_hbm.at[cid, pl.ds(_CK, _GP - _CK)])


def _mm_first_body(x_ref, w_ref, o_ref):
    o_ref[...] = jnp.dot(x_ref[...], w_ref[...], preferred_element_type=_f32)


def _mm_mid_body(p_ref, b_ref, w_ref, o_ref):
    h = jnp.maximum(p_ref[0] + p_ref[1] + b_ref[...], 0.0)
    o_ref[...] = jnp.dot(h, w_ref[...], preferred_element_type=_f32)


def _combine_body(p_ref, cinv_ref, b_ref, o_ref):
    o_ref[...] = (p_ref[0] + p_ref[1]) * cinv_ref[...] + b_ref[...]


@functools.lru_cache(maxsize=None)
def _build(n, e, d):
    ep = -((e + n) // -(_NW * _CK)) * (_NW * _CK)
    nc = ep // _NW // _CK
    assert nc % 3 == 0 and nc % 2 == 1, nc
    mesh = plsc.VectorSubcoreMesh(core_axis_name="c", subcore_axis_name="s")
    sc_params = pltpu.CompilerParams(needs_layout_passes=False,
                                     use_tc_tiling_on_sc=False)

    phase_p = pl.kernel(
        functools.partial(_phase_p_body, ep),
        out_type=(jax.ShapeDtypeStruct((ep // _CK, _CK), _f32),
                  jax.ShapeDtypeStruct((_GP * _D,), _f32)),
        mesh=mesh,
        scratch_types=[
            pltpu.VMEM_SHARED((_N_PAD,), _f32),
            pltpu.VMEM_SHARED((_GP,), _f32),
            pltpu.VMEM_SHARED((_N_PAD,), _f32),
            pltpu.VMEM((ep // 16 // _CK, _CK), _f32),
            pltpu.VMEM((ep // 16 // _CK, _CK), _i32),
            pltpu.VMEM((ep // _NW // _CK, _CK), _i32),
            pltpu.VMEM((ep // _NW // _CK, _CK), _i32),
            pltpu.VMEM((ep // _NW // _CK, _CK), _f32),
            pltpu.VMEM((_CK,), _f32),
            pltpu.VMEM((_BCK,), _i32),
            pltpu.VMEM((64,), _f32),
            pltpu.VMEM((64,), _f32),
            pltpu.VMEM((_N_PAD,), _f32),
            pltpu.VMEM((_CK,), _f32),
            pltpu.VMEM((_CK,), _f32),
            pltpu.VMEM((_GP,), _f32),
            pltpu.VMEM((_GP,), _f32),
            pltpu.VMEM((_GP * _D,), _f32),
            pltpu.SemaphoreType.DMA,
            pltpu.SemaphoreType.DMA,
            pltpu.SemaphoreType.DMA,
        ],
        name="gcn_phase_p",
        compiler_params=sc_params,
    )

    npt_chunks = ep // _CK // 16
    nc1 = -(-npt_chunks // 9) * 3
    nc0 = npt_chunks - nc1
    assert nc0 % 3 == 0 and nc1 % 3 == 0 and nc0 >= 3 and nc1 >= 3

    def make_prop(pool):
        return pl.kernel(
            functools.partial(_prop_body, ep, nc0, nc1, pool),
            out_type=jax.ShapeDtypeStruct(
                (2, _GP, _D) if pool else (2, _N_PAD, _D), _f32),
            mesh=mesh,
            scratch_types=[
                pltpu.VMEM_SHARED((_N_PAD, _D), _f32),
                pltpu.VMEM((_CK,), _i32),
                pltpu.VMEM((_CK,), _i32),
                pltpu.VMEM((_CK,), _i32),
                pltpu.VMEM((_CK,), _i32),
                pltpu.VMEM((_CK,), _i32),
                pltpu.VMEM((_CK,), _i32),
                pltpu.VMEM((_CK,), _f32),
                pltpu.VMEM((_CK,), _f32),
                pltpu.VMEM((_CK,), _f32),
                pltpu.VMEM((_CK, _D), _f32),
                pltpu.VMEM((_CK, _D), _f32),
                pltpu.VMEM((_CK, _D), _f32),
                pltpu.VMEM((_BCK,), _i32),
                pltpu.SemaphoreType.DMA,
                pltpu.SemaphoreType.DMA,
                pltpu.SemaphoreType.DMA,
                pltpu.SemaphoreType.DMA,
                pltpu.SemaphoreType.DMA,
                pltpu.SemaphoreType.DMA,
                pltpu.SemaphoreType.DMA,
                pltpu.SemaphoreType.DMA,
                pltpu.SemaphoreType.DMA,
            ] + ([pltpu.VMEM_SHARED((_GP, _D), _f32)] if pool else []),
            name="gcn_prop_pool" if pool else "gcn_prop",
            compiler_params=sc_params,
        )

    prop = make_prop(False)
    prop_pool = make_prop(True)

    nb = _N_PAD // 2048
    mm_first = pl.pallas_call(
        _mm_first_body,
        grid=(nb,),
        in_specs=[pl.BlockSpec((2048, d), lambda i: (i, 0)),
                  pl.BlockSpec((d, d), lambda i: (0, 0))],
        out_specs=pl.BlockSpec((2048, d), lambda i: (i, 0)),
        out_shape=jax.ShapeDtypeStruct((_N_PAD, d), _f32),
    )
    mm_mid = pl.pallas_call(
        _mm_mid_body,
        grid=(nb,),
        in_specs=[pl.BlockSpec((2, 2048, d), lambda i: (0, i, 0)),
                  pl.BlockSpec((1, d), lambda i: (0, 0)),
                  pl.BlockSpec((d, d), lambda i: (0, 0))],
        out_specs=pl.BlockSpec((2048, d), lambda i: (i, 0)),
        out_shape=jax.ShapeDtypeStruct((_N_PAD, d), _f32),
    )
    combine = pl.pallas_call(
        _combine_body,
        out_shape=jax.ShapeDtypeStruct((_GP, _D), _f32),
    )
    return ep, phase_p, prop, prop_pool, mm_first, mm_mid, combine


def kernel(x, edge_index, edge_weight, batch, batch_size,
           W_in, b_in, W_h0, b_h0, W_h1, b_h1, W_out, b_out):
    n, d = x.shape
    e = edge_index.shape[1]
    ep, phase_p, prop, prop_pool, mm_first, mm_mid, combine = _build(n, e, d)

    ar = jnp.arange(n, dtype=_i32)
    pad_e = ep - e - n
    row2d = jnp.concatenate([edge_index[0].astype(_i32), ar,
                             jnp.zeros((pad_e,), _i32)]).reshape(-1, _CK)
    col2d = jnp.concatenate([edge_index[1].astype(_i32), ar,
                             jnp.zeros((pad_e,), _i32)]).reshape(-1, _CK)
    ew2d = jnp.concatenate([edge_weight, jnp.ones((n,), _f32),
                            jnp.zeros((pad_e,), _f32)]).reshape(-1, _CK)
    batch2d = jnp.concatenate([batch.astype(_i32),
                               jnp.full((_N_PAD - n,), _GP - 1, _i32)
                               ]).reshape(-1, _BCK)
    x_p = jnp.zeros((_N_PAD, d), _f32).at[:n].set(x)
    z1 = jnp.zeros((_CK,), _f32)
    z = jnp.zeros((_CK, _D), _f32)

    norm2d, cinv = phase_p(row2d, col2d, ew2d, batch2d, z1)

    xw = mm_first(x_p, W_in)
    p = prop(xw, row2d, col2d, norm2d, z, batch2d)
    xw = mm_mid(p, b_in.reshape(1, d), W_h0)
    p = prop(xw, row2d, col2d, norm2d, z, batch2d)
    xw = mm_mid(p, b_h0.reshape(1, d), W_h1)
    p = prop(xw, row2d, col2d, norm2d, z, batch2d)
    xw = mm_mid(p, b_h1.reshape(1, d), W_out)
    pool = prop_pool(xw, row2d, col2d, norm2d, z, batch2d)

    out = combine(pool, cinv.reshape(_GP, _D), b_out.reshape(1, d))
    emb = out[:_G].reshape(_BS, _G // _BS, d)
    return emb + (jnp.asarray(batch_size) * 0).astype(emb.dtype)


# speedup vs baseline: 11.8033x; 1.0070x over previous
import functools

import jax
import jax.numpy as jnp
from jax import lax
from jax.experimental import pallas as pl
from jax.experimental.pallas import tpu as pltpu
from jax.experimental.pallas import tpu_sc as plsc

_i32 = jnp.int32
_f32 = jnp.float32

_D = 128
_G = 100
_BS = 10
_GP = 128
_N_PAD = 10240
_NW = 32
_CK = 112
_NPT = _N_PAD // 16
_BCK = 64


def _it16():
    return lax.iota(_i32, 16)


def _rsqrt16(d):
    i = plsc.bitcast(d, _i32)
    y = plsc.bitcast(jnp.full((16,), 0x5F3759DF, _i32) -
                     lax.shift_right_logical(i, 1), _f32)
    for _ in range(4):
        y = y * (1.5 - 0.5 * d * y * y)
    return jnp.where(d > 0.5, y, 0.0)


def _zero_chunks():
    off, out = 0, []
    while off < _NPT:
        sz = min(_CK, _NPT - off)
        out.append((off, sz))
        off += sz
    return out


def _phase_p_body(ep, row_hbm, col_hbm, ew_hbm, batch_hbm, z1_hbm,
                  norm_hbm, cinv_hbm,
                  deg_sh, cnt_sh, dinv_sh,
                  ewb_p, colb_p, rowb_n, colb_n, ewb_n,
                  ones, pidx, dtile, dchunk, dinv_full, nbuf0, nbuf1,
                  ctile, cinvb, cbc, dsem, ns0, ns1):
    cid = lax.axis_index("c")
    sid = lax.axis_index("s")

    pltpu.sync_copy(z1_hbm, nbuf0)
    for off, sz in _zero_chunks():
        pltpu.sync_copy(nbuf0.at[pl.ds(0, sz)],
                        deg_sh.at[pl.ds(sid * _NPT + off, sz)])
    for j in range(_CK // 16):
        ones[pl.ds(j * 16, 16)] = jnp.ones((16,), _f32)

    @pl.when(jnp.logical_and(cid == 0, sid < 8))
    def _():
        pltpu.sync_copy(nbuf0.at[pl.ds(0, 16)], cnt_sh.at[pl.ds(sid * 16, 16)])

    plsc.subcore_barrier()

    npc = ep // 16 // _CK
    pltpu.sync_copy(ew_hbm.at[pl.ds(sid * npc, npc)], ewb_p)
    pltpu.sync_copy(col_hbm.at[pl.ds(sid * npc, npc)], colb_p)

    def _dstart(c):
        pltpu.async_copy(ewb_p.at[c], deg_sh.at[colb_p.at[c]], dsem, add=True)

    def _dwait():
        pltpu.make_async_copy(ewb_p.at[0], deg_sh.at[colb_p.at[0]],
                              dsem).wait()

    _dstart(0)
    _dstart(1)

    def deg_chunk(c, _):
        _dwait()
        _dstart(c + 2)
        return 0

    lax.fori_loop(0, npc - 2, deg_chunk, 0)
    _dwait()
    _dwait()

    @pl.when(cid == 0)
    def _():
        for k in range(_NPT // _BCK):
            pltpu.sync_copy(batch_hbm.at[sid * (_NPT // _BCK) + k], pidx)
            pltpu.sync_copy(ones.at[pl.ds(0, _BCK)], cnt_sh.at[pidx],
                            add=True)

    plsc.subcore_barrier()

    def dinv_chunk(c, _):
        base = pl.multiple_of(sid * _NPT + c * 64, 64)
        pltpu.sync_copy(deg_sh.at[pl.ds(base, 64)], dtile)
        for j in range(4):
            d16 = dtile[pl.ds(j * 16, 16)]
            dchunk[pl.ds(j * 16, 16)] = _rsqrt16(d16)
        pltpu.sync_copy(dchunk, dinv_sh.at[pl.ds(base, 64)])
        return 0

    lax.fori_loop(0, _NPT // 64, dinv_chunk, 0)
    plsc.subcore_barrier()

    pltpu.sync_copy(dinv_sh, dinv_full)
    wid = sid * 2 + cid
    nc = ep // _NW // _CK
    pltpu.sync_copy(row_hbm.at[pl.ds(wid * nc, nc)], rowb_n)
    pltpu.sync_copy(col_hbm.at[pl.ds(wid * nc, nc)], colb_n)
    pltpu.sync_copy(ew_hbm.at[pl.ds(wid * nc, nc)], ewb_n)

    def _ncompute(c, buf):
        for j in range(_CK // 16):
            r16 = rowb_n[c, pl.ds(j * 16, 16)]
            c16 = colb_n[c, pl.ds(j * 16, 16)]
            w16 = ewb_n[c, pl.ds(j * 16, 16)]
            dr = plsc.load_gather(dinv_full, [r16])
            dc = plsc.load_gather(dinv_full, [c16])
            buf[pl.ds(j * 16, 16)] = dr * w16 * dc

    def _nstart(c, buf, sem):
        pltpu.async_copy(buf, norm_hbm.at[wid * nc + c], sem)

    def _nwait(buf, sem):
        pltpu.make_async_copy(buf, norm_hbm.at[0], sem).wait()

    def npair(cc, _):
        for par in range(2):
            c = 2 * cc + par
            buf, sem = (nbuf0, ns0) if par == 0 else (nbuf1, ns1)

            @pl.when(cc >= 1)
            def _(buf=buf, sem=sem):
                _nwait(buf, sem)

            _ncompute(c, buf)
            _nstart(c, buf, sem)
        return 0

    lax.fori_loop(0, (nc - 1) // 2, npair, 0)
    _nwait(nbuf0, ns0)
    _ncompute(nc - 1, nbuf0)
    _nstart(nc - 1, nbuf0, ns0)
    _nwait(nbuf0, ns0)
    _nwait(nbuf1, ns1)

    @pl.when(jnp.logical_and(cid == 0, sid == 0))
    def _():
        pltpu.sync_copy(cnt_sh, ctile)
        for j in range(8):
            c16 = ctile[pl.ds(j * 16, 16)]
            cinvb[pl.ds(j * 16, 16)] = 1.0 / jnp.maximum(c16, 1.0)

        def bc_row(r, _):
            s16 = plsc.load_gather(cinvb, [jnp.full((16,), r, _i32)])
            for v in range(8):
                cbc[pl.ds(r * _D + v * 16, 16)] = s16
            return 0

        lax.fori_loop(0, _GP, bc_row, 0)
        pltpu.sync_copy(cbc, cinv_hbm)


def _prop_body(ep, nc0, nc1, pool, xw_hbm, row_hbm, col_hbm, norm_hbm, z_hbm,
               batch_hbm, out_hbm, acc_sh,
               rb0, rb1, rb2, cb0, cb1, cb2, nb0, nb1, nb2,
               m0, m1, m2, pidx,
               is0, is1, is2, gs0, gs1, gs2, ss0, ss1, ss2, *maybe_pool):
    pool_sh = maybe_pool[0] if pool else None
    cid = lax.axis_index("c")
    sid = lax.axis_index("s")

    nc = jnp.where(cid == 0, nc0, nc1)
    cbase = jnp.where(cid == 0, sid * nc0, 16 * nc0 + sid * nc1)
    slots = ((rb0, cb0, nb0, m0, is0, gs0, ss0),
             (rb1, cb1, nb1, m1, is1, gs1, ss1),
             (rb2, cb2, nb2, m2, is2, gs2, ss2))

    def _istart(c, s):
        rb, cb, nb = s[0], s[1], s[2]
        pltpu.async_copy(row_hbm.at[cbase + c], rb, s[4])
        pltpu.async_copy(col_hbm.at[cbase + c], cb, s[4])
        pltpu.async_copy(norm_hbm.at[cbase + c], nb, s[4])

    def _iwait(s):
        for _ in range(3):
            pltpu.make_async_copy(row_hbm.at[0], s[0], s[4]).wait()

    def _gstart(c, s):
        pltpu.async_copy(xw_hbm.at[s[0]], s[3], s[5])

    def _gwait(s):
        pltpu.make_async_copy(xw_hbm.at[s[0]], s[3], s[5]).wait()

    def _sstart(c, s):
        pltpu.async_copy(s[3], acc_sh.at[s[1]], s[6], add=True)

    def _swait(s):
        pltpu.make_async_copy(s[3], acc_sh.at[s[1]], s[6]).wait()

    def _scale(s):
        nb, m = s[2], s[3]

        def sc(i, _):
            e = i * 2
            sa = plsc.load_gather(nb, [jnp.full((16,), e, _i32)])
            sb = plsc.load_gather(nb, [jnp.full((16,), e + 1, _i32)])
            for v in range(8):
                xa = m[e, pl.ds(v * 16, 16)]
                xb = m[e + 1, pl.ds(v * 16, 16)]
                m[e, pl.ds(v * 16, 16)] = xa * sa
                m[e + 1, pl.ds(v * 16, 16)] = xb * sb
            return 0

        lax.fori_loop(0, _CK // 2, sc, 0)

    pltpu.sync_copy(z_hbm, m0)
    _istart(0, slots[0])
    _istart(1, slots[1])
    _istart(2, slots[2])
    for off, sz in _zero_chunks():
        pltpu.async_copy(m0.at[pl.ds(0, sz)],
                         acc_sh.at[pl.ds(sid * _NPT + off, sz)], ss0)
    if pool:
        @pl.when(sid == 0)
        def _():
            pltpu.sync_copy(m0, pool_sh.at[pl.ds(0, _CK)])
            pltpu.sync_copy(m0.at[pl.ds(0, _GP - _CK)],
                            pool_sh.at[pl.ds(_CK, _GP - _CK)])
    for off, sz in _zero_chunks():
        pltpu.make_async_copy(m0.at[pl.ds(0, sz)],
                              acc_sh.at[pl.ds(0, sz)], ss0).wait()
    _iwait(slots[0])
    _gstart(0, slots[0])
    plsc.subcore_barrier()

    def triple(cc, _):
        for par in range(3):
            c = cc * 3 + par
            s = slots[par]
            sg = slots[(par + 1) % 3]
            sf = slots[(par + 2) % 3]
            _gwait(s)

            @pl.when(c <= nc - 2)
            def _(c=c, sg=sg):
                _iwait(sg)
                _gstart(c + 1, sg)

            _scale(s)

            @pl.when(jnp.logical_and(c >= 1, c <= nc - 3))
            def _(c=c, sf=sf):
                _swait(sf)
                _istart(c + 2, sf)

            _sstart(c, s)
        return 0

    lax.fori_loop(0, nc // 3, triple, 0)
    for s in slots:
        _swait(s)
    plsc.subcore_barrier()

    if not pool:
        chunks = _zero_chunks()
        for i, (off, sz) in enumerate(chunks):
            m = m0 if i % 2 == 0 else m1
            sem = gs0 if i % 2 == 0 else gs1
            if i >= 2:
                psz = chunks[i - 2][1]
                pltpu.make_async_copy(m.at[pl.ds(0, psz)],
                                      out_hbm.at[cid, pl.ds(0, psz)],
                                      sem).wait()
            base = sid * _NPT + off
            pltpu.sync_copy(acc_sh.at[pl.ds(base, sz)], m.at[pl.ds(0, sz)])
            pltpu.async_copy(m.at[pl.ds(0, sz)],
                             out_hbm.at[cid, pl.ds(base, sz)], sem)
        for i in (len(chunks) - 2, len(chunks) - 1):
            off, sz = chunks[i]
            m = m0 if i % 2 == 0 else m1
            sem = gs0 if i % 2 == 0 else gs1
            pltpu.make_async_copy(m.at[pl.ds(0, sz)],
                                  out_hbm.at[cid, pl.ds(0, sz)], sem).wait()
    else:
        for k in range(_NPT // _BCK):
            base = sid * _NPT + k * _BCK
            pltpu.sync_copy(acc_sh.at[pl.ds(base, _BCK)],
                            m0.at[pl.ds(0, _BCK)])
            pltpu.sync_copy(batch_hbm.at[sid * (_NPT // _BCK) + k], pidx)
            pltpu.sync_copy(m0.at[pl.ds(0, _BCK)], pool_sh.at[pidx],
                            add=True)
        plsc.subcore_barrier()

        @pl.when(sid == 0)
        def _():
            pltpu.sync_copy(pool_sh.at[pl.ds(0, _CK)], m0)
            pltpu.sync_copy(m0, out_hbm.at[cid, pl.ds(0, _CK)])
            pltpu.sync_copy(pool_sh.at[pl.ds(_CK, _GP - _CK)],
                            m0.at[pl.ds(0, _GP - _CK)])
            pltpu.sync_copy(m0.at[pl.ds(0, _GP - _CK)],
                            out_hbm.at[cid, pl.ds(_CK, _GP - _CK)])


def _mm_first_body(x_ref, w_ref, o_ref):
    o_ref[...] = jnp.dot(x_ref[...], w_ref[...], preferred_element_type=_f32)


def _mm_mid_body(p_ref, b_ref, w_ref, o_ref):
    h = jnp.maximum(p_ref[0] + p_ref[1] + b_ref[...], 0.0)
    o_ref[...] = jnp.dot(h, w_ref[...], preferred_element_type=_f32)


def _combine_body(p_ref, cinv_ref, b_ref, o_ref):
    o_ref[...] = (p_ref[0] + p_ref[1]) * cinv_ref[...] + b_ref[...]


@functools.lru_cache(maxsize=None)
def _build(n, e, d):
    ep = -((e + n) // -(_NW * _CK)) * (_NW * _CK)
    nc = ep // _NW // _CK
    assert nc % 3 == 0 and nc % 2 == 1, nc
    mesh = plsc.VectorSubcoreMesh(core_axis_name="c", subcore_axis_name="s")
    sc_params = pltpu.CompilerParams(needs_layout_passes=False,
                                     use_tc_tiling_on_sc=False)

    phase_p = pl.kernel(
        functools.partial(_phase_p_body, ep),
        out_type=(jax.ShapeDtypeStruct((ep // _CK, _CK), _f32),
                  jax.ShapeDtypeStruct((_GP * _D,), _f32)),
        mesh=mesh,
        scratch_types=[
            pltpu.VMEM_SHARED((_N_PAD,), _f32),
            pltpu.VMEM_SHARED((_GP,), _f32),
            pltpu.VMEM_SHARED((_N_PAD,), _f32),
            pltpu.VMEM((ep // 16 // _CK, _CK), _f32),
            pltpu.VMEM((ep // 16 // _CK, _CK), _i32),
            pltpu.VMEM((ep // _NW // _CK, _CK), _i32),
            pltpu.VMEM((ep // _NW // _CK, _CK), _i32),
            pltpu.VMEM((ep // _NW // _CK, _CK), _f32),
            pltpu.VMEM((_CK,), _f32),
            pltpu.VMEM((_BCK,), _i32),
            pltpu.VMEM((64,), _f32),
            pltpu.VMEM((64,), _f32),
            pltpu.VMEM((_N_PAD,), _f32),
            pltpu.VMEM((_CK,), _f32),
            pltpu.VMEM((_CK,), _f32),
            pltpu.VMEM((_GP,), _f32),
            pltpu.VMEM((_GP,), _f32),
            pltpu.VMEM((_GP * _D,), _f32),
            pltpu.SemaphoreType.DMA,
            pltpu.SemaphoreType.DMA,
            pltpu.SemaphoreType.DMA,
        ],
        name="gcn_phase_p",
        compiler_params=sc_params,
    )

    npt_chunks = ep // _CK // 16
    nc1 = -(-npt_chunks // 9) * 3
    nc0 = npt_chunks - nc1
    assert nc0 % 3 == 0 and nc1 % 3 == 0 and nc0 >= 3 and nc1 >= 3

    def make_prop(pool):
        return pl.kernel(
            functools.partial(_prop_body, ep, nc0, nc1, pool),
            out_type=jax.ShapeDtypeStruct(
                (2, _GP, _D) if pool else (2, _N_PAD, _D), _f32),
            mesh=mesh,
            scratch_types=[
                pltpu.VMEM_SHARED((_N_PAD, _D), _f32),
                pltpu.VMEM((_CK,), _i32),
                pltpu.VMEM((_CK,), _i32),
                pltpu.VMEM((_CK,), _i32),
                pltpu.VMEM((_CK,), _i32),
                pltpu.VMEM((_CK,), _i32),
                pltpu.VMEM((_CK,), _i32),
                pltpu.VMEM((_CK,), _f32),
                pltpu.VMEM((_CK,), _f32),
                pltpu.VMEM((_CK,), _f32),
                pltpu.VMEM((_CK, _D), _f32),
                pltpu.VMEM((_CK, _D), _f32),
                pltpu.VMEM((_CK, _D), _f32),
                pltpu.VMEM((_BCK,), _i32),
                pltpu.SemaphoreType.DMA,
                pltpu.SemaphoreType.DMA,
                pltpu.SemaphoreType.DMA,
                pltpu.SemaphoreType.DMA,
                pltpu.SemaphoreType.DMA,
                pltpu.SemaphoreType.DMA,
                pltpu.SemaphoreType.DMA,
                pltpu.SemaphoreType.DMA,
                pltpu.SemaphoreType.DMA,
            ] + ([pltpu.VMEM_SHARED((_GP, _D), _f32)] if pool else []),
            name="gcn_prop_pool" if pool else "gcn_prop",
            compiler_params=sc_params,
        )

    prop = make_prop(False)
    prop_pool = make_prop(True)

    nb = _N_PAD // 2048
    mm_first = pl.pallas_call(
        _mm_first_body,
        grid=(nb,),
        in_specs=[pl.BlockSpec((2048, d), lambda i: (i, 0)),
                  pl.BlockSpec((d, d), lambda i: (0, 0))],
        out_specs=pl.BlockSpec((2048, d), lambda i: (i, 0)),
        out_shape=jax.ShapeDtypeStruct((_N_PAD, d), _f32),
    )
    mm_mid = pl.pallas_call(
        _mm_mid_body,
        grid=(nb,),
        in_specs=[pl.BlockSpec((2, 2048, d), lambda i: (0, i, 0)),
                  pl.BlockSpec((1, d), lambda i: (0, 0)),
                  pl.BlockSpec((d, d), lambda i: (0, 0))],
        out_specs=pl.BlockSpec((2048, d), lambda i: (i, 0)),
        out_shape=jax.ShapeDtypeStruct((_N_PAD, d), _f32),
    )
    combine = pl.pallas_call(
        _combine_body,
        out_shape=jax.ShapeDtypeStruct((_GP, _D), _f32),
    )
    return ep, phase_p, prop, prop_pool, mm_first, mm_mid, combine


def kernel(x, edge_index, edge_weight, batch, batch_size,
           W_in, b_in, W_h0, b_h0, W_h1, b_h1, W_out, b_out):
    n, d = x.shape
    e = edge_index.shape[1]
    ep, phase_p, prop, prop_pool, mm_first, mm_mid, combine = _build(n, e, d)

    ar = jnp.arange(n, dtype=_i32)
    pad_e = ep - e - n
    row2d = jnp.concatenate([edge_index[0].astype(_i32), ar,
                             jnp.zeros((pad_e,), _i32)]).reshape(-1, _CK)
    col2d = jnp.concatenate([edge_index[1].astype(_i32), ar,
                             jnp.zeros((pad_e,), _i32)]).reshape(-1, _CK)
    ew2d = jnp.concatenate([edge_weight, jnp.ones((n,), _f32),
                            jnp.zeros((pad_e,), _f32)]).reshape(-1, _CK)
    batch2d = jnp.concatenate([batch.astype(_i32),
                               jnp.full((_N_PAD - n,), _GP - 1, _i32)
                               ]).reshape(-1, _BCK)
    x_p = jnp.zeros((_N_PAD, d), _f32).at[:n].set(x)
    z1 = jnp.zeros((_CK,), _f32)
    z = jnp.zeros((_CK, _D), _f32)

    norm2d, cinv = phase_p(row2d, col2d, ew2d, batch2d, z1)

    xw = mm_first(x_p, W_in)
    p = prop(xw, row2d, col2d, norm2d, z, batch2d)
    xw = mm_mid(p, b_in.reshape(1, d), W_h0)
    p = prop(xw, row2d, col2d, norm2d, z, batch2d)
    xw = mm_mid(p, b_h0.reshape(1, d), W_h1)
    p = prop(xw, row2d, col2d, norm2d, z, batch2d)
    xw = mm_mid(p, b_h1.reshape(1, d), W_out)
    pool = prop_pool(xw, row2d, col2d, norm2d, z, batch2d)

    out = combine(pool, cinv.reshape(_GP, _D), b_out.reshape(1, d))
    emb = out[:_G].reshape(_BS, _G // _BS, d)
    return emb + (jnp.asarray(batch_size) * 0).astype(emb.dtype)


# speedup vs baseline: 13.0148x; 1.1026x over previous
import functools

import jax
import jax.numpy as jnp
from jax import lax
from jax.experimental import pallas as pl
from jax.experimental.pallas import tpu as pltpu
from jax.experimental.pallas import tpu_sc as plsc

_i32 = jnp.int32
_f32 = jnp.float32

_D = 128
_G = 100
_BS = 10
_GP = 128
_N_PAD = 10240
_NW = 32
_CK = 112
_NPT = _N_PAD // 16
_BCK = 64


def _it16():
    return lax.iota(_i32, 16)


def _rsqrt16(d):
    i = plsc.bitcast(d, _i32)
    y = plsc.bitcast(jnp.full((16,), 0x5F3759DF, _i32) -
                     lax.shift_right_logical(i, 1), _f32)
    for _ in range(4):
        y = y * (1.5 - 0.5 * d * y * y)
    return jnp.where(d > 0.5, y, 0.0)


def _zero_chunks():
    off, out = 0, []
    while off < _NPT:
        sz = min(_CK, _NPT - off)
        out.append((off, sz))
        off += sz
    return out


def _phase_p_body(ep, row_hbm, col_hbm, ew_hbm, batch_hbm, z1_hbm,
                  norm_hbm, cinv_hbm,
                  deg_sh, cnt_sh, dinv_sh,
                  ewb_p, colb_p, rowb_n, colb_n, ewb_n,
                  ones, pidx, dtile, dchunk, dinv_full, nbuf0, nbuf1,
                  ctile, cinvb, cbc, dsem, ns0, ns1):
    cid = lax.axis_index("c")
    sid = lax.axis_index("s")

    pltpu.sync_copy(z1_hbm, nbuf0)
    for off, sz in _zero_chunks():
        pltpu.sync_copy(nbuf0.at[pl.ds(0, sz)],
                        deg_sh.at[pl.ds(sid * _NPT + off, sz)])
    for j in range(_CK // 16):
        ones[pl.ds(j * 16, 16)] = jnp.ones((16,), _f32)

    @pl.when(jnp.logical_and(cid == 0, sid < 8))
    def _():
        pltpu.sync_copy(nbuf0.at[pl.ds(0, 16)], cnt_sh.at[pl.ds(sid * 16, 16)])

    plsc.subcore_barrier()

    npc = ep // 16 // _CK
    pltpu.sync_copy(ew_hbm.at[pl.ds(sid * npc, npc)], ewb_p)
    pltpu.sync_copy(col_hbm.at[pl.ds(sid * npc, npc)], colb_p)

    def _dstart(c):
        pltpu.async_copy(ewb_p.at[c], deg_sh.at[colb_p.at[c]], dsem, add=True)

    def _dwait():
        pltpu.make_async_copy(ewb_p.at[0], deg_sh.at[colb_p.at[0]],
                              dsem).wait()

    _dstart(0)
    _dstart(1)

    def deg_chunk(c, _):
        _dwait()
        _dstart(c + 2)
        return 0

    lax.fori_loop(0, npc - 2, deg_chunk, 0)
    _dwait()
    _dwait()

    @pl.when(cid == 0)
    def _():
        for k in range(_NPT // _BCK):
            pltpu.sync_copy(batch_hbm.at[sid * (_NPT // _BCK) + k], pidx)
            pltpu.sync_copy(ones.at[pl.ds(0, _BCK)], cnt_sh.at[pidx],
                            add=True)

    plsc.subcore_barrier()

    def dinv_chunk(c, _):
        base = pl.multiple_of(sid * _NPT + c * 64, 64)
        pltpu.sync_copy(deg_sh.at[pl.ds(base, 64)], dtile)
        for j in range(4):
            d16 = dtile[pl.ds(j * 16, 16)]
            dchunk[pl.ds(j * 16, 16)] = _rsqrt16(d16)
        pltpu.sync_copy(dchunk, dinv_sh.at[pl.ds(base, 64)])
        return 0

    lax.fori_loop(0, _NPT // 64, dinv_chunk, 0)
    plsc.subcore_barrier()

    pltpu.sync_copy(dinv_sh, dinv_full)
    wid = sid * 2 + cid
    nc = ep // _NW // _CK
    pltpu.sync_copy(row_hbm.at[pl.ds(wid * nc, nc)], rowb_n)
    pltpu.sync_copy(col_hbm.at[pl.ds(wid * nc, nc)], colb_n)
    pltpu.sync_copy(ew_hbm.at[pl.ds(wid * nc, nc)], ewb_n)

    def _ncompute(c, buf):
        for j in range(_CK // 16):
            r16 = rowb_n[c, pl.ds(j * 16, 16)]
            c16 = colb_n[c, pl.ds(j * 16, 16)]
            w16 = ewb_n[c, pl.ds(j * 16, 16)]
            dr = plsc.load_gather(dinv_full, [r16])
            dc = plsc.load_gather(dinv_full, [c16])
            buf[pl.ds(j * 16, 16)] = dr * w16 * dc

    def _nstart(c, buf, sem):
        pltpu.async_copy(buf, norm_hbm.at[wid * nc + c], sem)

    def _nwait(buf, sem):
        pltpu.make_async_copy(buf, norm_hbm.at[0], sem).wait()

    def npair(cc, _):
        for par in range(2):
            c = 2 * cc + par
            buf, sem = (nbuf0, ns0) if par == 0 else (nbuf1, ns1)

            @pl.when(cc >= 1)
            def _(buf=buf, sem=sem):
                _nwait(buf, sem)

            _ncompute(c, buf)
            _nstart(c, buf, sem)
        return 0

    lax.fori_loop(0, (nc - 1) // 2, npair, 0)
    _nwait(nbuf0, ns0)
    _ncompute(nc - 1, nbuf0)
    _nstart(nc - 1, nbuf0, ns0)
    _nwait(nbuf0, ns0)
    _nwait(nbuf1, ns1)

    @pl.when(jnp.logical_and(cid == 0, sid == 0))
    def _():
        pltpu.sync_copy(cnt_sh, ctile)
        for j in range(8):
            c16 = ctile[pl.ds(j * 16, 16)]
            cinvb[pl.ds(j * 16, 16)] = 1.0 / jnp.maximum(c16, 1.0)

        def bc_row(r, _):
            s16 = plsc.load_gather(cinvb, [jnp.full((16,), r, _i32)])
            for v in range(8):
                cbc[pl.ds(r * _D + v * 16, 16)] = s16
            return 0

        lax.fori_loop(0, _GP, bc_row, 0)
        pltpu.sync_copy(cbc, cinv_hbm)


def _prop_body(ep, nc0, nc1, pool, xw_hbm, row_hbm, col_hbm, norm_hbm, z_hbm,
               batch_hbm, out_hbm, acc_sh,
               rb0, rb1, rb2, cb0, cb1, cb2, nb0, nb1, nb2,
               m0, m1, m2, pidx,
               is0, is1, is2, gs0, gs1, gs2, ss0, ss1, ss2, *maybe_pool):
    pool_sh = maybe_pool[0] if pool else None
    cid = lax.axis_index("c")
    sid = lax.axis_index("s")

    nc = jnp.where(cid == 0, nc0, nc1)
    cbase = jnp.where(cid == 0, sid * nc0, 16 * nc0 + sid * nc1)
    slots = ((rb0, cb0, nb0, m0, is0, gs0, ss0),
             (rb1, cb1, nb1, m1, is1, gs1, ss1),
             (rb2, cb2, nb2, m2, is2, gs2, ss2))

    def _istart(c, s):
        rb, cb, nb = s[0], s[1], s[2]
        pltpu.async_copy(row_hbm.at[cbase + c], rb, s[4])
        pltpu.async_copy(col_hbm.at[cbase + c], cb, s[4])
        pltpu.async_copy(norm_hbm.at[cbase + c], nb, s[4])

    def _iwait(s):
        for _ in range(3):
            pltpu.make_async_copy(row_hbm.at[0], s[0], s[4]).wait()

    def _gstart(c, s):
        pltpu.async_copy(xw_hbm.at[s[0]], s[3], s[5])

    def _gwait(s):
        pltpu.make_async_copy(xw_hbm.at[s[0]], s[3], s[5]).wait()

    def _sstart(c, s):
        pltpu.async_copy(s[3], acc_sh.at[s[1]], s[6], add=True)

    def _swait(s):
        pltpu.make_async_copy(s[3], acc_sh.at[s[1]], s[6]).wait()

    def _scale(s):
        nb, m = s[2], s[3]

        def sc(i, _):
            e = i * 2
            sa = plsc.load_gather(nb, [jnp.full((16,), e, _i32)])
            sb = plsc.load_gather(nb, [jnp.full((16,), e + 1, _i32)])
            for v in range(8):
                xa = m[e, pl.ds(v * 16, 16)]
                xb = m[e + 1, pl.ds(v * 16, 16)]
                m[e, pl.ds(v * 16, 16)] = xa * sa
                m[e + 1, pl.ds(v * 16, 16)] = xb * sb
            return 0

        lax.fori_loop(0, _CK // 2, sc, 0)

    pltpu.sync_copy(z_hbm, m0)
    _istart(0, slots[0])
    _istart(1, slots[1])
    _istart(2, slots[2])
    for off, sz in _zero_chunks():
        pltpu.async_copy(m0.at[pl.ds(0, sz)],
                         acc_sh.at[pl.ds(sid * _NPT + off, sz)], ss0)
    if pool:
        @pl.when(sid == 0)
        def _():
            pltpu.sync_copy(m0, pool_sh.at[pl.ds(0, _CK)])
            pltpu.sync_copy(m0.at[pl.ds(0, _GP - _CK)],
                            pool_sh.at[pl.ds(_CK, _GP - _CK)])
    for off, sz in _zero_chunks():
        pltpu.make_async_copy(m0.at[pl.ds(0, sz)],
                              acc_sh.at[pl.ds(0, sz)], ss0).wait()
    _iwait(slots[0])
    _gstart(0, slots[0])
    plsc.subcore_barrier()

    def triple(cc, _):
        for par in range(3):
            c = cc * 3 + par
            s = slots[par]
            sg = slots[(par + 1) % 3]
            sf = slots[(par + 2) % 3]
            _gwait(s)

            @pl.when(c <= nc - 2)
            def _(c=c, sg=sg):
                _iwait(sg)
                _gstart(c + 1, sg)

            _scale(s)

            @pl.when(jnp.logical_and(c >= 1, c <= nc - 3))
            def _(c=c, sf=sf):
                _swait(sf)
                _istart(c + 2, sf)

            _sstart(c, s)
        return 0

    lax.fori_loop(0, nc // 3, triple, 0)
    for s in slots:
        _swait(s)
    plsc.subcore_barrier()

    if not pool:
        chunks = _zero_chunks()
        for i, (off, sz) in enumerate(chunks):
            m = m0 if i % 2 == 0 else m1
            sem = gs0 if i % 2 == 0 else gs1
            if i >= 2:
                psz = chunks[i - 2][1]
                pltpu.make_async_copy(m.at[pl.ds(0, psz)],
                                      out_hbm.at[cid, pl.ds(0, psz)],
                                      sem).wait()
            base = sid * _NPT + off
            pltpu.sync_copy(acc_sh.at[pl.ds(base, sz)], m.at[pl.ds(0, sz)])
            pltpu.async_copy(m.at[pl.ds(0, sz)],
                             out_hbm.at[cid, pl.ds(base, sz)], sem)
        for i in (len(chunks) - 2, len(chunks) - 1):
            off, sz = chunks[i]
            m = m0 if i % 2 == 0 else m1
            sem = gs0 if i % 2 == 0 else gs1
            pltpu.make_async_copy(m.at[pl.ds(0, sz)],
                                  out_hbm.at[cid, pl.ds(0, sz)], sem).wait()
    else:
        for k in range(_NPT // _BCK):
            base = sid * _NPT + k * _BCK
            pltpu.sync_copy(acc_sh.at[pl.ds(base, _BCK)],
                            m0.at[pl.ds(0, _BCK)])
            pltpu.sync_copy(batch_hbm.at[sid * (_NPT // _BCK) + k], pidx)
            pltpu.sync_copy(m0.at[pl.ds(0, _BCK)], pool_sh.at[pidx],
                            add=True)
        plsc.subcore_barrier()

        @pl.when(sid == 0)
        def _():
            pltpu.sync_copy(pool_sh.at[pl.ds(0, _CK)], m0)
            pltpu.sync_copy(m0, out_hbm.at[cid, pl.ds(0, _CK)])
            pltpu.sync_copy(pool_sh.at[pl.ds(_CK, _GP - _CK)],
                            m0.at[pl.ds(0, _GP - _CK)])
            pltpu.sync_copy(m0.at[pl.ds(0, _GP - _CK)],
                            out_hbm.at[cid, pl.ds(_CK, _GP - _CK)])


def _mm_first_body(x_ref, w_ref, o_ref):
    o_ref[...] = jnp.dot(x_ref[...], w_ref[...], preferred_element_type=_f32)


def _mm_mid_body(p_ref, b_ref, w_ref, o_ref):
    h = jnp.maximum(p_ref[0] + p_ref[1] + b_ref[...], 0.0)
    o_ref[...] = jnp.dot(h, w_ref[...], preferred_element_type=_f32)


def _combine_body(p_ref, cinv_ref, b_ref, o_ref):
    o_ref[...] = (p_ref[0] + p_ref[1]) * cinv_ref[...] + b_ref[...]


@functools.lru_cache(maxsize=None)
def _build(n, e, d):
    ep = -((e + n) // -(_NW * _CK)) * (_NW * _CK)
    nc = ep // _NW // _CK
    assert nc % 3 == 0 and nc % 2 == 1, nc
    mesh = plsc.VectorSubcoreMesh(core_axis_name="c", subcore_axis_name="s")
    sc_params = pltpu.CompilerParams(needs_layout_passes=False,
                                     use_tc_tiling_on_sc=False)

    phase_p = pl.kernel(
        functools.partial(_phase_p_body, ep),
        out_type=(jax.ShapeDtypeStruct((ep // _CK, _CK), _f32),
                  jax.ShapeDtypeStruct((_GP * _D,), _f32)),
        mesh=mesh,
        scratch_types=[
            pltpu.VMEM_SHARED((_N_PAD,), _f32),
            pltpu.VMEM_SHARED((_GP,), _f32),
            pltpu.VMEM_SHARED((_N_PAD,), _f32),
            pltpu.VMEM((ep // 16 // _CK, _CK), _f32),
            pltpu.VMEM((ep // 16 // _CK, _CK), _i32),
            pltpu.VMEM((ep // _NW // _CK, _CK), _i32),
            pltpu.VMEM((ep // _NW // _CK, _CK), _i32),
            pltpu.VMEM((ep // _NW // _CK, _CK), _f32),
            pltpu.VMEM((_CK,), _f32),
            pltpu.VMEM((_BCK,), _i32),
            pltpu.VMEM((64,), _f32),
            pltpu.VMEM((64,), _f32),
            pltpu.VMEM((_N_PAD,), _f32),
            pltpu.VMEM((_CK,), _f32),
            pltpu.VMEM((_CK,), _f32),
            pltpu.VMEM((_GP,), _f32),
            pltpu.VMEM((_GP,), _f32),
            pltpu.VMEM((_GP * _D,), _f32),
            pltpu.SemaphoreType.DMA,
            pltpu.SemaphoreType.DMA,
            pltpu.SemaphoreType.DMA,
        ],
        name="gcn_phase_p",
        compiler_params=sc_params,
    )

    npt_chunks = ep // _CK // 16
    nc1 = -(-npt_chunks * 21 // 100 // 3) * 3
    nc0 = npt_chunks - nc1
    assert nc0 % 3 == 0 and nc1 % 3 == 0 and nc0 >= 3 and nc1 >= 3

    def make_prop(pool):
        return pl.kernel(
            functools.partial(_prop_body, ep, nc0, nc1, pool),
            out_type=jax.ShapeDtypeStruct(
                (2, _GP, _D) if pool else (2, _N_PAD, _D), _f32),
            mesh=mesh,
            scratch_types=[
                pltpu.VMEM_SHARED((_N_PAD, _D), _f32),
                pltpu.VMEM((_CK,), _i32),
                pltpu.VMEM((_CK,), _i32),
                pltpu.VMEM((_CK,), _i32),
                pltpu.VMEM((_CK,), _i32),
                pltpu.VMEM((_CK,), _i32),
                pltpu.VMEM((_CK,), _i32),
                pltpu.VMEM((_CK,), _f32),
                pltpu.VMEM((_CK,), _f32),
                pltpu.VMEM((_CK,), _f32),
                pltpu.VMEM((_CK, _D), _f32),
                pltpu.VMEM((_CK, _D), _f32),
                pltpu.VMEM((_CK, _D), _f32),
                pltpu.VMEM((_BCK,), _i32),
                pltpu.SemaphoreType.DMA,
                pltpu.SemaphoreType.DMA,
                pltpu.SemaphoreType.DMA,
                pltpu.SemaphoreType.DMA,
                pltpu.SemaphoreType.DMA,
                pltpu.SemaphoreType.DMA,
                pltpu.SemaphoreType.DMA,
                pltpu.SemaphoreType.DMA,
                pltpu.SemaphoreType.DMA,
            ] + ([pltpu.VMEM_SHARED((_GP, _D), _f32)] if pool else []),
            name="gcn_prop_pool" if pool else "gcn_prop",
            compiler_params=sc_params,
        )

    prop = make_prop(False)
    prop_pool = make_prop(True)

    nb = _N_PAD // 2048
    mm_first = pl.pallas_call(
        _mm_first_body,
        grid=(nb,),
        in_specs=[pl.BlockSpec((2048, d), lambda i: (i, 0)),
                  pl.BlockSpec((d, d), lambda i: (0, 0))],
        out_specs=pl.BlockSpec((2048, d), lambda i: (i, 0)),
        out_shape=jax.ShapeDtypeStruct((_N_PAD, d), _f32),
    )
    mm_mid = pl.pallas_call(
        _mm_mid_body,
        grid=(nb,),
        in_specs=[pl.BlockSpec((2, 2048, d), lambda i: (0, i, 0)),
                  pl.BlockSpec((1, d), lambda i: (0, 0)),
                  pl.BlockSpec((d, d), lambda i: (0, 0))],
        out_specs=pl.BlockSpec((2048, d), lambda i: (i, 0)),
        out_shape=jax.ShapeDtypeStruct((_N_PAD, d), _f32),
    )
    combine = pl.pallas_call(
        _combine_body,
        out_shape=jax.ShapeDtypeStruct((_GP, _D), _f32),
    )
    return ep, phase_p, prop, prop_pool, mm_first, mm_mid, combine


def kernel(x, edge_index, edge_weight, batch, batch_size,
           W_in, b_in, W_h0, b_h0, W_h1, b_h1, W_out, b_out):
    n, d = x.shape
    e = edge_index.shape[1]
    ep, phase_p, prop, prop_pool, mm_first, mm_mid, combine = _build(n, e, d)

    ar = jnp.arange(n, dtype=_i32)
    pad_e = ep - e - n
    row2d = jnp.concatenate([edge_index[0].astype(_i32), ar,
                             jnp.zeros((pad_e,), _i32)]).reshape(-1, _CK)
    col2d = jnp.concatenate([edge_index[1].astype(_i32), ar,
                             jnp.zeros((pad_e,), _i32)]).reshape(-1, _CK)
    ew2d = jnp.concatenate([edge_weight, jnp.ones((n,), _f32),
                            jnp.zeros((pad_e,), _f32)]).reshape(-1, _CK)
    batch2d = jnp.concatenate([batch.astype(_i32),
                               jnp.full((_N_PAD - n,), _GP - 1, _i32)
                               ]).reshape(-1, _BCK)
    x_p = jnp.zeros((_N_PAD, d), _f32).at[:n].set(x)
    z1 = jnp.zeros((_CK,), _f32)
    z = jnp.zeros((_CK, _D), _f32)

    norm2d, cinv = phase_p(row2d, col2d, ew2d, batch2d, z1)

    xw = mm_first(x_p, W_in)
    p = prop(xw, row2d, col2d, norm2d, z, batch2d)
    xw = mm_mid(p, b_in.reshape(1, d), W_h0)
    p = prop(xw, row2d, col2d, norm2d, z, batch2d)
    xw = mm_mid(p, b_h0.reshape(1, d), W_h1)
    p = prop(xw, row2d, col2d, norm2d, z, batch2d)
    xw = mm_mid(p, b_h1.reshape(1, d), W_out)
    pool = prop_pool(xw, row2d, col2d, norm2d, z, batch2d)

    out = combine(pool, cinv.reshape(_GP, _D), b_out.reshape(1, d))
    emb = out[:_G].reshape(_BS, _G // _BS, d)
    return emb + (jnp.asarray(batch_size) * 0).astype(emb.dtype)
